# baseline (speedup 1.0000x reference)
import functools

import jax
import jax.numpy as jnp
from jax import lax
from jax.experimental import pallas as pl
from jax.experimental.pallas import tpu as pltpu
from jax.experimental.pallas import tpu_sc as plsc

N = 10000
NE = 320000
D = 128
V = 42
H = 64
EBLK = 8000
NBLK = 2000
EPS = 1e-5


def _swish(x):
    return x * (0.5 * jnp.tanh(0.5 * x) + 0.5)


def _pad8(b):
    return jnp.zeros((8, H), jnp.float32).at[0].set(b)


def _proj_body(x_ref, wa_ref, wb_ref, a_ref, b_ref):
    x = x_ref[...]
    a_ref[...] = jnp.dot(x, wa_ref[...], preferred_element_type=jnp.float32)
    b_ref[...] = jnp.dot(x, wb_ref[...], preferred_element_type=jnp.float32)


def _proj(x2, w0a, w0b):
    g = N // NBLK
    return pl.pallas_call(
        _proj_body,
        grid=(g,),
        in_specs=[
            pl.BlockSpec((NBLK, D), lambda i: (i, 0)),
            pl.BlockSpec((D, H), lambda i: (0, 0)),
            pl.BlockSpec((D, H), lambda i: (0, 0)),
        ],
        out_specs=[
            pl.BlockSpec((NBLK, H), lambda i: (i, 0)),
            pl.BlockSpec((NBLK, H), lambda i: (i, 0)),
        ],
        out_shape=[
            jax.ShapeDtypeStruct((N, H), jnp.float32),
            jax.ShapeDtypeStruct((N, H), jnp.float32),
        ],
    )(x2, w0a, w0b)


def _msg_body(ga_ref, gb_ref, b0_ref, gbe_ref, w1_ref, w2_ref, w3_ref,
              b123_ref, x_ref, m_ref, acc, nrm, wsel):
    p = pl.program_id(0)
    i = pl.program_id(1)

    @pl.when((p == 0) & (i == 0))
    def _():
        acc[...] = jnp.zeros_like(acc)

    @pl.when((p >= 1) & (i == 0))
    def _():
        mu = acc[0:1, :] * (1.0 / NE)
        var = acc[1:2, :] * (1.0 / NE) - mu * mu
        g = jnp.where(p == 1, gbe_ref[0:1, :],
                      jnp.where(p == 2, gbe_ref[2:3, :], gbe_ref[4:5, :]))
        be = jnp.where(p == 1, gbe_ref[1:2, :],
                       jnp.where(p == 2, gbe_ref[3:4, :], gbe_ref[5:6, :]))
        s = g * lax.rsqrt(var + EPS)
        nrm[0:1, :] = s
        nrm[1:2, :] = be - mu * s
        nrm[2:3, :] = jnp.where(p == 1, b123_ref[0:1, :],
                                jnp.where(p == 2, b123_ref[1:2, :],
                                          b123_ref[2:3, :]))
        wsel[...] = jnp.where(p == 1, w1_ref[...],
                              jnp.where(p == 2, w2_ref[...], w3_ref[...]))
        acc[...] = jnp.zeros_like(acc)

    @pl.when(p == 0)
    def _():
        h = _swish(ga_ref[...] + gb_ref[...] + b0_ref[0:1, :])
        x_ref[...] = h
        acc[0:1, :] = acc[0:1, :] + jnp.sum(h, axis=0, keepdims=True)
        acc[1:2, :] = acc[1:2, :] + jnp.sum(h * h, axis=0, keepdims=True)

    @pl.when(p >= 1)
    def _():
        xn = ga_ref[...] * nrm[0:1, :] + nrm[1:2, :]
        y = _swish(
            jnp.dot(xn, wsel[...], preferred_element_type=jnp.float32)
            + nrm[2:3, :]
        )

        @pl.when(p < 3)
        def _():
            x_ref[...] = y
            acc[0:1, :] = acc[0:1, :] + jnp.sum(y, axis=0, keepdims=True)
            acc[1:2, :] = acc[1:2, :] + jnp.sum(y * y, axis=0, keepdims=True)

        @pl.when(p == 3)
        def _():
            m_ref[...] = y


def _msgnet(ga, gb, msg):
    ge = NE // EBLK
    b0p = _pad8(msg['b'][0])
    gbe = jnp.zeros((8, H), jnp.float32)
    gbe = gbe.at[0].set(msg['g'][0]).at[1].set(msg['be'][0])
    gbe = gbe.at[2].set(msg['g'][1]).at[3].set(msg['be'][1])
    gbe = gbe.at[4].set(msg['g'][2]).at[5].set(msg['be'][2])
    b123 = (jnp.zeros((8, H), jnp.float32)
            .at[0].set(msg['b'][1]).at[1].set(msg['b'][2]).at[2].set(msg['b'][3]))
    cst = lambda shape: pl.BlockSpec(shape, lambda p, i: tuple(0 for _ in shape))
    _, m = pl.pallas_call(
        _msg_body,
        grid=(4, ge),
        in_specs=[
            pl.BlockSpec((EBLK, H), lambda p, i: (i, 0)),
            pl.BlockSpec((EBLK, H), lambda p, i: (jnp.where(p == 0, i, 0), 0)),
            cst((8, H)), cst((8, H)),
            cst((H, H)), cst((H, H)), cst((H, H)), cst((8, H)),
        ],
        out_specs=[
            pl.BlockSpec((EBLK, H), lambda p, i: (jnp.where(p < 3, i, 0), 0)),
            pl.BlockSpec((EBLK, H), lambda p, i: (jnp.where(p == 3, i, 0), 0)),
        ],
        out_shape=[
            jax.ShapeDtypeStruct((NE, H), jnp.float32),
            jax.ShapeDtypeStruct((NE, H), jnp.float32),
        ],
        scratch_shapes=[
            pltpu.VMEM((8, H), jnp.float32),
            pltpu.VMEM((8, H), jnp.float32),
            pltpu.VMEM((H, H), jnp.float32),
        ],
        input_output_aliases={0: 0},
    )(ga, gb, b0p, gbe, msg['W'][1], msg['W'][2], msg['W'][3], b123)
    return m


def _upd_body(x1_ref, x2_ref, s2_ref, c2_ref,
              wa_ref, wb_ref, wc_ref, b0_ref, w1_ref, b1_ref,
              w2_ref, b2_ref, w3_ref, b3_ref, xo_ref):
    cnt = c2_ref[0, :, 0:1] + c2_ref[1, :, 0:1]
    cnt = jnp.maximum(cnt, 1.0)
    agg = (s2_ref[0] + s2_ref[1]) / cnt
    t = _swish(
        jnp.dot(x1_ref[...], wa_ref[...], preferred_element_type=jnp.float32)
        + jnp.dot(x2_ref[...], wb_ref[...], preferred_element_type=jnp.float32)
        + jnp.dot(agg, wc_ref[...], preferred_element_type=jnp.float32)
        + b0_ref[0:1, :]
    )
    t = _swish(jnp.dot(t, w1_ref[...], preferred_element_type=jnp.float32) + b1_ref[0:1, :])
    t = _swish(jnp.dot(t, w2_ref[...], preferred_element_type=jnp.float32) + b2_ref[0:1, :])
    t = _swish(jnp.dot(t, w3_ref[...], preferred_element_type=jnp.float32) + b3_ref[0:1, :])
    xo_ref[...] = x2_ref[...] + t


def _upd(x1p, x2, s2, c2, upd_p):
    wu0 = upd_p['W'][0]
    wa = jnp.zeros((H, H), jnp.float32).at[:V].set(wu0[:V])
    wb = wu0[V:V + D]
    wc = wu0[V + D:]
    g = N // NBLK
    cst = lambda shape: pl.BlockSpec(shape, lambda i: tuple(0 for _ in shape))
    return pl.pallas_call(
        _upd_body,
        grid=(g,),
        in_specs=[
            pl.BlockSpec((NBLK, H), lambda i: (i, 0)),
            pl.BlockSpec((NBLK, D), lambda i: (i, 0)),
            pl.BlockSpec((2, NBLK, H), lambda i: (0, i, 0)),
            pl.BlockSpec((2, NBLK, 16), lambda i: (0, i, 0)),
            cst((H, H)), cst((D, H)), cst((H, H)), cst((8, H)),
            cst((H, H)), cst((8, H)),
            cst((H, H)), cst((8, H)),
            cst((H, D)), cst((8, D)),
        ],
        out_specs=pl.BlockSpec((NBLK, D), lambda i: (i, 0)),
        out_shape=jax.ShapeDtypeStruct((N, D), jnp.float32),
    )(x1p, x2, s2, c2,
      wa, wb, wc, _pad8(upd_p['b'][0]),
      upd_p['W'][1], _pad8(upd_p['b'][1]),
      upd_p['W'][2], _pad8(upd_p['b'][2]),
      upd_p['W'][3],
      jnp.zeros((8, D), jnp.float32).at[0].set(upd_p['b'][3]))


def _final_body(x2_ref, x1h_ref,
                wp0_ref, bp0_ref, wp1_ref, bp1_ref, wp2_ref, bp2_ref,
                wp3_ref, bp3_ref, wq0_ref, bq0_ref, wq1_ref, bq1_ref,
                out_ref, acc):
    i = pl.program_id(0)
    t = _swish(jnp.dot(x2_ref[...], wp0_ref[...], preferred_element_type=jnp.float32) + bp0_ref[0:1, :])
    t = _swish(jnp.dot(t, wp1_ref[...], preferred_element_type=jnp.float32) + bp1_ref[0:1, :])
    t = _swish(jnp.dot(t, wp2_ref[...], preferred_element_type=jnp.float32) + bp2_ref[0:1, :])
    h = jnp.dot(t, wp3_ref[...], preferred_element_type=jnp.float32) + bp3_ref[0:1, :]

    @pl.when(i == 0)
    def _():
        acc[...] = jnp.zeros_like(acc)

    acc[0:1, :] = acc[0:1, :] + jnp.sum(h, axis=0, keepdims=True)

    @pl.when(i == pl.num_programs(0) - 1)
    def _():
        pooled = acc[...] / N
        c = _swish(jnp.dot(pooled, wq0_ref[...], preferred_element_type=jnp.float32) + bq0_ref[0:1, :])
        coeff = jnp.dot(c, wq1_ref[...], preferred_element_type=jnp.float32) + bq1_ref[0:1, :]
        out_ref[...] = jnp.dot(coeff, x1h_ref[...], preferred_element_type=jnp.float32)


def _final(x2, x1h, pre_p, post_p):
    g = N // NBLK
    cst = lambda shape: pl.BlockSpec(shape, lambda i: tuple(0 for _ in shape))
    return pl.pallas_call(
        _final_body,
        grid=(g,),
        in_specs=[
            pl.BlockSpec((NBLK, D), lambda i: (i, 0)),
            cst((H, H)),
            cst((D, H)), cst((8, H)),
            cst((H, H)), cst((8, H)),
            cst((H, H)), cst((8, H)),
            cst((H, H)), cst((8, H)),
            cst((H, H)), cst((8, H)),
            cst((H, H)), cst((8, H)),
        ],
        out_specs=pl.BlockSpec((8, H), lambda i: (0, 0)),
        out_shape=jax.ShapeDtypeStruct((8, H), jnp.float32),
        scratch_shapes=[pltpu.VMEM((8, H), jnp.float32)],
    )(x2, x1h,
      pre_p['W'][0], _pad8(pre_p['b'][0]),
      pre_p['W'][1], _pad8(pre_p['b'][1]),
      pre_p['W'][2], _pad8(pre_p['b'][2]),
      pre_p['W'][3], _pad8(pre_p['b'][3]),
      post_p['W'][0], _pad8(post_p['b'][0]),
      post_p['W'][1], _pad8(post_p['b'][1]))


NC = 2
NS = 16
NW = NC * NS
EW = NE // NW
GC = 400


_SC_PARAMS = pltpu.CompilerParams(use_tc_tiling_on_sc=False)


def _gather(a, b, dst, src):
    mesh = plsc.VectorSubcoreMesh(core_axis_name="c", subcore_axis_name="s")

    @functools.partial(
        pl.kernel,
        mesh=mesh,
        out_type=[
            jax.ShapeDtypeStruct((NE, H), jnp.float32),
            jax.ShapeDtypeStruct((NE, H), jnp.float32),
        ],
        scratch_types=[
            pltpu.VMEM((GC,), jnp.int32),
            pltpu.VMEM((GC,), jnp.int32),
            pltpu.VMEM((GC, H), jnp.float32),
            pltpu.VMEM((GC, H), jnp.float32),
            pltpu.SemaphoreType.DMA,
            pltpu.SemaphoreType.DMA,
        ],
        compiler_params=_SC_PARAMS,
    )
    def k(a_hbm, b_hbm, dst_hbm, src_hbm, ga_hbm, gb_hbm, idxd, idxs,
          rowd, rows, sema, semb):
        wid = lax.axis_index("s") * NC + lax.axis_index("c")
        base = wid * EW

        def body(j, carry):
            e0 = base + j * GC
            pltpu.sync_copy(dst_hbm.at[pl.ds(e0, GC)], idxd)
            pltpu.sync_copy(src_hbm.at[pl.ds(e0, GC)], idxs)
            cpa = pltpu.async_copy(a_hbm.at[idxd], rowd, sema)
            cpb = pltpu.async_copy(b_hbm.at[idxs], rows, semb)
            cpa.wait()
            cpb.wait()
            pltpu.sync_copy(rowd, ga_hbm.at[pl.ds(e0, GC)])
            pltpu.sync_copy(rows, gb_hbm.at[pl.ds(e0, GC)])
            return carry

        lax.fori_loop(0, EW // GC, body, 0)

    return k(a, b, dst, src)


def _scatter(m, dst):
    mesh = plsc.VectorSubcoreMesh(core_axis_name="c", subcore_axis_name="s")
    z64 = jnp.zeros((N, H), jnp.float32)
    z16 = jnp.zeros((N, 16), jnp.float32)
    ones = jnp.ones((GC, 16), jnp.float32)
    nrows = N // NS

    @functools.partial(
        pl.kernel,
        mesh=mesh,
        out_type=[
            jax.ShapeDtypeStruct((NC, N, H), jnp.float32),
            jax.ShapeDtypeStruct((NC, N, 16), jnp.float32),
        ],
        scratch_types=[
            pltpu.VMEM((GC,), jnp.int32),
            pltpu.VMEM((GC, H), jnp.float32),
            pltpu.VMEM((GC, 16), jnp.float32),
            pltpu.VMEM_SHARED((N, H), jnp.float32),
            pltpu.VMEM_SHARED((N, 16), jnp.float32),
        ],
        compiler_params=_SC_PARAMS,
    )
    def k(m_hbm, dst_hbm, z64_hbm, z16_hbm, ones_hbm, s_hbm, c_hbm,
          idx, rows, onev, acc, accc):
        cid = lax.axis_index("c")
        sid = lax.axis_index("s")
        wid = sid * NC + cid
        base = wid * EW
        pltpu.sync_copy(ones_hbm, onev)

        @pl.when(sid == 0)
        def _():
            pltpu.sync_copy(z64_hbm, acc)
            pltpu.sync_copy(z16_hbm, accc)

        plsc.subcore_barrier()

        def body(j, carry):
            e0 = base + j * GC
            pltpu.sync_copy(dst_hbm.at[pl.ds(e0, GC)], idx)
            pltpu.sync_copy(m_hbm.at[pl.ds(e0, GC)], rows)
            pltpu.sync_copy(rows, acc.at[idx], add=True)
            pltpu.sync_copy(onev, accc.at[idx], add=True)
            return carry

        lax.fori_loop(0, EW // GC, body, 0)
        plsc.subcore_barrier()
        r0 = sid * nrows
        pltpu.sync_copy(acc.at[pl.ds(r0, nrows)], s_hbm.at[cid, pl.ds(r0, nrows)])
        pltpu.sync_copy(accc.at[pl.ds(r0, nrows)], c_hbm.at[cid, pl.ds(r0, nrows)])

    return k(m, dst, z64, z16, ones)


def kernel(node_feature, vectors, params, edge_index):
    x0 = node_feature[0]
    src = edge_index[0, 0]
    dst = edge_index[0, 1]
    x1 = x0[:, :V]
    x1p = jnp.pad(x1, ((0, 0), (0, H - V)))
    x1h = jnp.pad(x1[:H], ((0, 0), (0, H - V)))

    x2 = x0
    for lp in params['gnn']:
        msg = lp['msg']
        w0 = msg['W'][0]
        w0a = w0[:D]
        w0b = jnp.zeros((D, H), jnp.float32).at[:V].set(w0[D:])
        a, b = _proj(x2, w0a, w0b)
        ga, gb = _gather(a, b, dst, src)
        m = _msgnet(ga, gb, msg)
        s2, c2 = _scatter(m, dst)
        x2 = _upd(x1p, x2, s2, c2, lp['upd'])

    out = _final(x2, x1h, params['pre'], params['post'])
    return out[0, :V]


# speedup vs baseline: 1.9372x; 1.9372x over previous
import functools

import jax
import jax.numpy as jnp
from jax import lax
from jax.experimental import pallas as pl
from jax.experimental.pallas import tpu as pltpu
from jax.experimental.pallas import tpu_sc as plsc

N = 10000
NE = 320000
D = 128
V = 42
H = 64
EBLK = 8000
NBLK = 2000
EPS = 1e-5


def _swish(x):
    return x * (0.5 * jnp.tanh(0.5 * x) + 0.5)


def _pad8(b):
    return jnp.zeros((8, H), jnp.float32).at[0].set(b)


def _proj_body(x_ref, wa_ref, wb_ref, a_ref, b_ref):
    x = x_ref[...]
    a_ref[...] = jnp.dot(x, wa_ref[...], preferred_element_type=jnp.float32)
    b_ref[...] = jnp.dot(x, wb_ref[...], preferred_element_type=jnp.float32)


def _proj(x2, w0a, w0b):
    g = N // NBLK
    return pl.pallas_call(
        _proj_body,
        grid=(g,),
        in_specs=[
            pl.BlockSpec((NBLK, D), lambda i: (i, 0)),
            pl.BlockSpec((D, H), lambda i: (0, 0)),
            pl.BlockSpec((D, H), lambda i: (0, 0)),
        ],
        out_specs=[
            pl.BlockSpec((NBLK, H), lambda i: (i, 0)),
            pl.BlockSpec((NBLK, H), lambda i: (i, 0)),
        ],
        out_shape=[
            jax.ShapeDtypeStruct((N, H), jnp.float32),
            jax.ShapeDtypeStruct((N, H), jnp.float32),
        ],
    )(x2, w0a, w0b)


NE2 = NE // 2
EBLK2 = EBLK // 2
D2 = 2 * H


def _msg_body(ga_ref, gb_ref, b0_ref, gbe_ref, w1_ref, w2_ref, w3_ref,
              b123_ref, psum_ref, x_ref, m_ref, acc, nrm, wsel):
    p = pl.program_id(0)
    i = pl.program_id(1)

    @pl.when((p == 0) & (i == 0))
    def _():
        acc[...] = jnp.zeros_like(acc)

    @pl.when((p >= 1) & (i == 0))
    def _():
        mu = jnp.dot(acc[0:1, :], psum_ref[...],
                     preferred_element_type=jnp.float32) * (1.0 / NE)
        msq = jnp.dot(acc[1:2, :], psum_ref[...],
                      preferred_element_type=jnp.float32) * (1.0 / NE)
        var = msq - mu * mu
        g = jnp.where(p == 1, gbe_ref[0:1, :],
                      jnp.where(p == 2, gbe_ref[2:3, :], gbe_ref[4:5, :]))
        be = jnp.where(p == 1, gbe_ref[1:2, :],
                       jnp.where(p == 2, gbe_ref[3:4, :], gbe_ref[5:6, :]))
        s = g * lax.rsqrt(var + EPS)
        nrm[0:1, :] = s
        nrm[1:2, :] = be - mu * s
        nrm[2:3, :] = jnp.where(p == 1, b123_ref[0:1, :],
                                jnp.where(p == 2, b123_ref[1:2, :],
                                          b123_ref[2:3, :]))
        wsel[...] = jnp.where(p == 1, w1_ref[...],
                              jnp.where(p == 2, w2_ref[...], w3_ref[...]))
        acc[...] = jnp.zeros_like(acc)

    @pl.when(p == 0)
    def _():
        h = _swish(ga_ref[...] + gb_ref[...] + b0_ref[0:1, :])
        x_ref[...] = h
        acc[0:1, :] = acc[0:1, :] + jnp.sum(h, axis=0, keepdims=True)
        acc[1:2, :] = acc[1:2, :] + jnp.sum(h * h, axis=0, keepdims=True)

    @pl.when(p >= 1)
    def _():
        xn = ga_ref[...] * nrm[0:1, :] + nrm[1:2, :]
        y = _swish(
            jnp.dot(xn, wsel[...], preferred_element_type=jnp.float32)
            + nrm[2:3, :]
        )

        @pl.when(p < 3)
        def _():
            x_ref[...] = y
            acc[0:1, :] = acc[0:1, :] + jnp.sum(y, axis=0, keepdims=True)
            acc[1:2, :] = acc[1:2, :] + jnp.sum(y * y, axis=0, keepdims=True)

        @pl.when(p == 3)
        def _():
            m_ref[...] = y


def _dup(v):
    return jnp.concatenate([v, v])


def _pad8d(rows):
    out = jnp.zeros((8, D2), jnp.float32)
    for r, v in enumerate(rows):
        out = out.at[r].set(_dup(v))
    return out


def _bdiag(w):
    return (jnp.zeros((D2, D2), jnp.float32)
            .at[:H, :H].set(w).at[H:, H:].set(w))


def _msgnet(ga, gb, msg):
    ge = NE2 // EBLK2
    b0p = _pad8d([msg['b'][0]])
    gbe = _pad8d([msg['g'][0], msg['be'][0], msg['g'][1], msg['be'][1],
                  msg['g'][2], msg['be'][2]])
    b123 = _pad8d([msg['b'][1], msg['b'][2], msg['b'][3]])
    eye = jnp.eye(H, dtype=jnp.float32)
    psum = jnp.block([[eye, eye], [eye, eye]])
    cst = lambda shape: pl.BlockSpec(shape, lambda p, i: tuple(0 for _ in shape))
    _, m = pl.pallas_call(
        _msg_body,
        grid=(4, ge),
        in_specs=[
            pl.BlockSpec((EBLK2, D2), lambda p, i: (i, 0)),
            pl.BlockSpec((EBLK2, D2), lambda p, i: (jnp.where(p == 0, i, 0), 0)),
            cst((8, D2)), cst((8, D2)),
            cst((D2, D2)), cst((D2, D2)), cst((D2, D2)), cst((8, D2)),
            cst((D2, D2)),
        ],
        out_specs=[
            pl.BlockSpec((EBLK2, D2), lambda p, i: (jnp.where(p < 3, i, 0), 0)),
            pl.BlockSpec((EBLK2, D2), lambda p, i: (jnp.where(p == 3, i, 0), 0)),
        ],
        out_shape=[
            jax.ShapeDtypeStruct((NE2, D2), jnp.float32),
            jax.ShapeDtypeStruct((NE2, D2), jnp.float32),
        ],
        scratch_shapes=[
            pltpu.VMEM((8, D2), jnp.float32),
            pltpu.VMEM((8, D2), jnp.float32),
            pltpu.VMEM((D2, D2), jnp.float32),
        ],
        input_output_aliases={0: 0},
    )(ga, gb, b0p, gbe, _bdiag(msg['W'][1]), _bdiag(msg['W'][2]),
      _bdiag(msg['W'][3]), b123, psum)
    return m


def _upd_body(x1_ref, x2_ref, s2_ref, c2_ref,
              wa_ref, wb_ref, wc_ref, b0_ref, w1_ref, b1_ref,
              w2_ref, b2_ref, w3_ref, b3_ref, xo_ref):
    cnt = c2_ref[0, :, 0:1] + c2_ref[1, :, 0:1]
    cnt = jnp.maximum(cnt, 1.0)
    agg = (s2_ref[0] + s2_ref[1]) / cnt
    t = _swish(
        jnp.dot(x1_ref[...], wa_ref[...], preferred_element_type=jnp.float32)
        + jnp.dot(x2_ref[...], wb_ref[...], preferred_element_type=jnp.float32)
        + jnp.dot(agg, wc_ref[...], preferred_element_type=jnp.float32)
        + b0_ref[0:1, :]
    )
    t = _swish(jnp.dot(t, w1_ref[...], preferred_element_type=jnp.float32) + b1_ref[0:1, :])
    t = _swish(jnp.dot(t, w2_ref[...], preferred_element_type=jnp.float32) + b2_ref[0:1, :])
    t = _swish(jnp.dot(t, w3_ref[...], preferred_element_type=jnp.float32) + b3_ref[0:1, :])
    xo_ref[...] = x2_ref[...] + t


def _upd(x1p, x2, s2, c2, upd_p):
    wu0 = upd_p['W'][0]
    wa = jnp.zeros((H, H), jnp.float32).at[:V].set(wu0[:V])
    wb = wu0[V:V + D]
    wc = wu0[V + D:]
    g = N // NBLK
    cst = lambda shape: pl.BlockSpec(shape, lambda i: tuple(0 for _ in shape))
    return pl.pallas_call(
        _upd_body,
        grid=(g,),
        in_specs=[
            pl.BlockSpec((NBLK, H), lambda i: (i, 0)),
            pl.BlockSpec((NBLK, D), lambda i: (i, 0)),
            pl.BlockSpec((2, NBLK, H), lambda i: (0, i, 0)),
            pl.BlockSpec((2, NBLK, 16), lambda i: (0, i, 0)),
            cst((H, H)), cst((D, H)), cst((H, H)), cst((8, H)),
            cst((H, H)), cst((8, H)),
            cst((H, H)), cst((8, H)),
            cst((H, D)), cst((8, D)),
        ],
        out_specs=pl.BlockSpec((NBLK, D), lambda i: (i, 0)),
        out_shape=jax.ShapeDtypeStruct((N, D), jnp.float32),
    )(x1p, x2, s2, c2,
      wa, wb, wc, _pad8(upd_p['b'][0]),
      upd_p['W'][1], _pad8(upd_p['b'][1]),
      upd_p['W'][2], _pad8(upd_p['b'][2]),
      upd_p['W'][3],
      jnp.zeros((8, D), jnp.float32).at[0].set(upd_p['b'][3]))


def _final_body(x2_ref, x1h_ref,
                wp0_ref, bp0_ref, wp1_ref, bp1_ref, wp2_ref, bp2_ref,
                wp3_ref, bp3_ref, wq0_ref, bq0_ref, wq1_ref, bq1_ref,
                out_ref, acc):
    i = pl.program_id(0)
    t = _swish(jnp.dot(x2_ref[...], wp0_ref[...], preferred_element_type=jnp.float32) + bp0_ref[0:1, :])
    t = _swish(jnp.dot(t, wp1_ref[...], preferred_element_type=jnp.float32) + bp1_ref[0:1, :])
    t = _swish(jnp.dot(t, wp2_ref[...], preferred_element_type=jnp.float32) + bp2_ref[0:1, :])
    h = jnp.dot(t, wp3_ref[...], preferred_element_type=jnp.float32) + bp3_ref[0:1, :]

    @pl.when(i == 0)
    def _():
        acc[...] = jnp.zeros_like(acc)

    acc[0:1, :] = acc[0:1, :] + jnp.sum(h, axis=0, keepdims=True)

    @pl.when(i == pl.num_programs(0) - 1)
    def _():
        pooled = acc[...] / N
        c = _swish(jnp.dot(pooled, wq0_ref[...], preferred_element_type=jnp.float32) + bq0_ref[0:1, :])
        coeff = jnp.dot(c, wq1_ref[...], preferred_element_type=jnp.float32) + bq1_ref[0:1, :]
        out_ref[...] = jnp.dot(coeff, x1h_ref[...], preferred_element_type=jnp.float32)


def _final(x2, x1h, pre_p, post_p):
    g = N // NBLK
    cst = lambda shape: pl.BlockSpec(shape, lambda i: tuple(0 for _ in shape))
    return pl.pallas_call(
        _final_body,
        grid=(g,),
        in_specs=[
            pl.BlockSpec((NBLK, D), lambda i: (i, 0)),
            cst((H, H)),
            cst((D, H)), cst((8, H)),
            cst((H, H)), cst((8, H)),
            cst((H, H)), cst((8, H)),
            cst((H, H)), cst((8, H)),
            cst((H, H)), cst((8, H)),
            cst((H, H)), cst((8, H)),
        ],
        out_specs=pl.BlockSpec((8, H), lambda i: (0, 0)),
        out_shape=jax.ShapeDtypeStruct((8, H), jnp.float32),
        scratch_shapes=[pltpu.VMEM((8, H), jnp.float32)],
    )(x2, x1h,
      pre_p['W'][0], _pad8(pre_p['b'][0]),
      pre_p['W'][1], _pad8(pre_p['b'][1]),
      pre_p['W'][2], _pad8(pre_p['b'][2]),
      pre_p['W'][3], _pad8(pre_p['b'][3]),
      post_p['W'][0], _pad8(post_p['b'][0]),
      post_p['W'][1], _pad8(post_p['b'][1]))


NC = 2
NS = 16
NW = NC * NS
EW = NE // NW
GC = 400


_SC_PARAMS = pltpu.CompilerParams(use_tc_tiling_on_sc=False)


def _gather(a, b, dst, src):
    mesh = plsc.VectorSubcoreMesh(core_axis_name="c", subcore_axis_name="s")

    @functools.partial(
        pl.kernel,
        mesh=mesh,
        out_type=[
            jax.ShapeDtypeStruct((NE, H), jnp.float32),
            jax.ShapeDtypeStruct((NE, H), jnp.float32),
        ],
        scratch_types=[
            pltpu.VMEM((GC,), jnp.int32),
            pltpu.VMEM((GC,), jnp.int32),
            pltpu.VMEM((GC, H), jnp.float32),
            pltpu.VMEM((GC, H), jnp.float32),
            pltpu.SemaphoreType.DMA,
            pltpu.SemaphoreType.DMA,
        ],
        compiler_params=_SC_PARAMS,
    )
    def k(a_hbm, b_hbm, dst_hbm, src_hbm, ga_hbm, gb_hbm, idxd, idxs,
          rowd, rows, sema, semb):
        wid = lax.axis_index("s") * NC + lax.axis_index("c")
        base = wid * EW

        def body(j, carry):
            e0 = base + j * GC
            pltpu.sync_copy(dst_hbm.at[pl.ds(e0, GC)], idxd)
            pltpu.sync_copy(src_hbm.at[pl.ds(e0, GC)], idxs)
            cpa = pltpu.async_copy(a_hbm.at[idxd], rowd, sema)
            cpb = pltpu.async_copy(b_hbm.at[idxs], rows, semb)
            cpa.wait()
            cpb.wait()
            pltpu.sync_copy(rowd, ga_hbm.at[pl.ds(e0, GC)])
            pltpu.sync_copy(rows, gb_hbm.at[pl.ds(e0, GC)])
            return carry

        lax.fori_loop(0, EW // GC, body, 0)

    return k(a, b, dst, src)


def _scatter(m, dst):
    mesh = plsc.VectorSubcoreMesh(core_axis_name="c", subcore_axis_name="s")
    z64 = jnp.zeros((N, H), jnp.float32)
    z16 = jnp.zeros((N, 16), jnp.float32)
    ones = jnp.ones((GC, 16), jnp.float32)
    nrows = N // NS

    @functools.partial(
        pl.kernel,
        mesh=mesh,
        out_type=[
            jax.ShapeDtypeStruct((NC, N, H), jnp.float32),
            jax.ShapeDtypeStruct((NC, N, 16), jnp.float32),
        ],
        scratch_types=[
            pltpu.VMEM((GC,), jnp.int32),
            pltpu.VMEM((GC, H), jnp.float32),
            pltpu.VMEM((GC, 16), jnp.float32),
            pltpu.VMEM_SHARED((N, H), jnp.float32),
            pltpu.VMEM_SHARED((N, 16), jnp.float32),
        ],
        compiler_params=_SC_PARAMS,
    )
    def k(m_hbm, dst_hbm, z64_hbm, z16_hbm, ones_hbm, s_hbm, c_hbm,
          idx, rows, onev, acc, accc):
        cid = lax.axis_index("c")
        sid = lax.axis_index("s")
        wid = sid * NC + cid
        base = wid * EW
        pltpu.sync_copy(ones_hbm, onev)

        @pl.when(sid == 0)
        def _():
            pltpu.sync_copy(z64_hbm, acc)
            pltpu.sync_copy(z16_hbm, accc)

        plsc.subcore_barrier()

        def body(j, carry):
            e0 = base + j * GC
            pltpu.sync_copy(dst_hbm.at[pl.ds(e0, GC)], idx)
            pltpu.sync_copy(m_hbm.at[pl.ds(e0, GC)], rows)
            pltpu.sync_copy(rows, acc.at[idx], add=True)
            pltpu.sync_copy(onev, accc.at[idx], add=True)
            return carry

        lax.fori_loop(0, EW // GC, body, 0)
        plsc.subcore_barrier()
        r0 = sid * nrows
        pltpu.sync_copy(acc.at[pl.ds(r0, nrows)], s_hbm.at[cid, pl.ds(r0, nrows)])
        pltpu.sync_copy(accc.at[pl.ds(r0, nrows)], c_hbm.at[cid, pl.ds(r0, nrows)])

    return k(m, dst, z64, z16, ones)


def kernel(node_feature, vectors, params, edge_index):
    x0 = node_feature[0]
    src = edge_index[0, 0]
    dst = edge_index[0, 1]
    x1 = x0[:, :V]
    x1p = jnp.pad(x1, ((0, 0), (0, H - V)))
    x1h = jnp.pad(x1[:H], ((0, 0), (0, H - V)))

    x2 = x0
    for lp in params['gnn']:
        msg = lp['msg']
        w0 = msg['W'][0]
        w0a = w0[:D]
        w0b = jnp.zeros((D, H), jnp.float32).at[:V].set(w0[D:])
        a, b = _proj(x2, w0a, w0b)
        ga, gb = _gather(a, b, dst, src)
        m = _msgnet(ga.reshape(NE2, D2), gb.reshape(NE2, D2), msg)
        s2, c2 = _scatter(m.reshape(NE, H), dst)
        x2 = _upd(x1p, x2, s2, c2, lp['upd'])

    out = _final(x2, x1h, params['pre'], params['post'])
    return out[0, :V]


# speedup vs baseline: 2.0217x; 1.0436x over previous
import functools

import jax
import jax.numpy as jnp
from jax import lax
from jax.experimental import pallas as pl
from jax.experimental.pallas import tpu as pltpu
from jax.experimental.pallas import tpu_sc as plsc

N = 10000
NE = 320000
D = 128
V = 42
H = 64
EBLK = 8000
NBLK = 2000
EPS = 1e-5


def _swish(x):
    return x * (0.5 * jnp.tanh(0.5 * x) + 0.5)


def _pad8(b):
    return jnp.zeros((8, H), jnp.float32).at[0].set(b)


def _proj_body(x_ref, wa_ref, wb_ref, a_ref, b_ref):
    x = x_ref[...]
    a_ref[...] = jnp.dot(x, wa_ref[...], preferred_element_type=jnp.float32)
    b_ref[...] = jnp.dot(x, wb_ref[...], preferred_element_type=jnp.float32)


def _proj(x2, w0a, w0b):
    g = N // NBLK
    return pl.pallas_call(
        _proj_body,
        grid=(g,),
        in_specs=[
            pl.BlockSpec((NBLK, D), lambda i: (i, 0)),
            pl.BlockSpec((D, H), lambda i: (0, 0)),
            pl.BlockSpec((D, H), lambda i: (0, 0)),
        ],
        out_specs=[
            pl.BlockSpec((NBLK, H), lambda i: (i, 0)),
            pl.BlockSpec((NBLK, H), lambda i: (i, 0)),
        ],
        out_shape=[
            jax.ShapeDtypeStruct((N, H), jnp.float32),
            jax.ShapeDtypeStruct((N, H), jnp.float32),
        ],
    )(x2, w0a, w0b)


NE2 = NE // 2
EBLK2 = EBLK // 2
D2 = 2 * H


def _msg_body(ga_ref, gb_ref, b0_ref, gbe_ref, w1_ref, w2_ref, w3_ref,
              b123_ref, psum_ref, x_ref, m_ref, acc, nrm, wsel):
    p = pl.program_id(0)
    i = pl.program_id(1)

    @pl.when((p == 0) & (i == 0))
    def _():
        acc[...] = jnp.zeros_like(acc)

    @pl.when((p >= 1) & (i == 0))
    def _():
        mu = jnp.dot(acc[0:1, :], psum_ref[...],
                     preferred_element_type=jnp.float32) * (1.0 / NE)
        msq = jnp.dot(acc[1:2, :], psum_ref[...],
                      preferred_element_type=jnp.float32) * (1.0 / NE)
        var = msq - mu * mu
        g = jnp.where(p == 1, gbe_ref[0:1, :],
                      jnp.where(p == 2, gbe_ref[2:3, :], gbe_ref[4:5, :]))
        be = jnp.where(p == 1, gbe_ref[1:2, :],
                       jnp.where(p == 2, gbe_ref[3:4, :], gbe_ref[5:6, :]))
        s = g * lax.rsqrt(var + EPS)
        nrm[0:1, :] = s
        nrm[1:2, :] = be - mu * s
        nrm[2:3, :] = jnp.where(p == 1, b123_ref[0:1, :],
                                jnp.where(p == 2, b123_ref[1:2, :],
                                          b123_ref[2:3, :]))
        wsel[...] = jnp.where(p == 1, w1_ref[...],
                              jnp.where(p == 2, w2_ref[...], w3_ref[...]))
        acc[...] = jnp.zeros_like(acc)

    @pl.when(p == 0)
    def _():
        h = _swish(ga_ref[...] + gb_ref[...] + b0_ref[0:1, :])
        x_ref[...] = h
        acc[0:1, :] = acc[0:1, :] + jnp.sum(h, axis=0, keepdims=True)
        acc[1:2, :] = acc[1:2, :] + jnp.sum(h * h, axis=0, keepdims=True)

    @pl.when(p >= 1)
    def _():
        xn = ga_ref[...] * nrm[0:1, :] + nrm[1:2, :]
        y = _swish(
            jnp.dot(xn, wsel[...], preferred_element_type=jnp.float32)
            + nrm[2:3, :]
        )

        @pl.when(p < 3)
        def _():
            x_ref[...] = y
            acc[0:1, :] = acc[0:1, :] + jnp.sum(y, axis=0, keepdims=True)
            acc[1:2, :] = acc[1:2, :] + jnp.sum(y * y, axis=0, keepdims=True)

        @pl.when(p == 3)
        def _():
            m_ref[...] = y


def _dup(v):
    return jnp.concatenate([v, v])


def _pad8d(rows):
    out = jnp.zeros((8, D2), jnp.float32)
    for r, v in enumerate(rows):
        out = out.at[r].set(_dup(v))
    return out


def _bdiag(w):
    return (jnp.zeros((D2, D2), jnp.float32)
            .at[:H, :H].set(w).at[H:, H:].set(w))


def _msgnet(ga, gb, msg):
    ge = NE2 // EBLK2
    b0p = _pad8d([msg['b'][0]])
    gbe = _pad8d([msg['g'][0], msg['be'][0], msg['g'][1], msg['be'][1],
                  msg['g'][2], msg['be'][2]])
    b123 = _pad8d([msg['b'][1], msg['b'][2], msg['b'][3]])
    eye = jnp.eye(H, dtype=jnp.float32)
    psum = jnp.block([[eye, eye], [eye, eye]])
    cst = lambda shape: pl.BlockSpec(shape, lambda p, i: tuple(0 for _ in shape))
    _, m = pl.pallas_call(
        _msg_body,
        grid=(4, ge),
        in_specs=[
            pl.BlockSpec((EBLK2, D2), lambda p, i: (i, 0)),
            pl.BlockSpec((EBLK2, D2), lambda p, i: (jnp.where(p == 0, i, 0), 0)),
            cst((8, D2)), cst((8, D2)),
            cst((D2, D2)), cst((D2, D2)), cst((D2, D2)), cst((8, D2)),
            cst((D2, D2)),
        ],
        out_specs=[
            pl.BlockSpec((EBLK2, D2), lambda p, i: (jnp.where(p < 3, i, 0), 0)),
            pl.BlockSpec((EBLK2, D2), lambda p, i: (jnp.where(p == 3, i, 0), 0)),
        ],
        out_shape=[
            jax.ShapeDtypeStruct((NE2, D2), jnp.float32),
            jax.ShapeDtypeStruct((NE2, D2), jnp.float32),
        ],
        scratch_shapes=[
            pltpu.VMEM((8, D2), jnp.float32),
            pltpu.VMEM((8, D2), jnp.float32),
            pltpu.VMEM((D2, D2), jnp.float32),
        ],
        input_output_aliases={0: 0},
    )(ga, gb, b0p, gbe, _bdiag(msg['W'][1]), _bdiag(msg['W'][2]),
      _bdiag(msg['W'][3]), b123, psum)
    return m


def _upd_body(x1_ref, x2_ref, s2_ref, c2_ref,
              wa_ref, wb_ref, wc_ref, b0_ref, w1_ref, b1_ref,
              w2_ref, b2_ref, w3_ref, b3_ref, xo_ref):
    cnt = c2_ref[0, :, 0:1] + c2_ref[1, :, 0:1]
    cnt = jnp.maximum(cnt, 1.0)
    agg = (s2_ref[0] + s2_ref[1]) / cnt
    t = _swish(
        jnp.dot(x1_ref[...], wa_ref[...], preferred_element_type=jnp.float32)
        + jnp.dot(x2_ref[...], wb_ref[...], preferred_element_type=jnp.float32)
        + jnp.dot(agg, wc_ref[...], preferred_element_type=jnp.float32)
        + b0_ref[0:1, :]
    )
    t = _swish(jnp.dot(t, w1_ref[...], preferred_element_type=jnp.float32) + b1_ref[0:1, :])
    t = _swish(jnp.dot(t, w2_ref[...], preferred_element_type=jnp.float32) + b2_ref[0:1, :])
    t = _swish(jnp.dot(t, w3_ref[...], preferred_element_type=jnp.float32) + b3_ref[0:1, :])
    xo_ref[...] = x2_ref[...] + t


def _upd(x1p, x2, s2, c2, upd_p):
    wu0 = upd_p['W'][0]
    wa = jnp.zeros((H, H), jnp.float32).at[:V].set(wu0[:V])
    wb = wu0[V:V + D]
    wc = wu0[V + D:]
    g = N // NBLK
    cst = lambda shape: pl.BlockSpec(shape, lambda i: tuple(0 for _ in shape))
    return pl.pallas_call(
        _upd_body,
        grid=(g,),
        in_specs=[
            pl.BlockSpec((NBLK, H), lambda i: (i, 0)),
            pl.BlockSpec((NBLK, D), lambda i: (i, 0)),
            pl.BlockSpec((2, NBLK, H), lambda i: (0, i, 0)),
            pl.BlockSpec((2, NBLK, 16), lambda i: (0, i, 0)),
            cst((H, H)), cst((D, H)), cst((H, H)), cst((8, H)),
            cst((H, H)), cst((8, H)),
            cst((H, H)), cst((8, H)),
            cst((H, D)), cst((8, D)),
        ],
        out_specs=pl.BlockSpec((NBLK, D), lambda i: (i, 0)),
        out_shape=jax.ShapeDtypeStruct((N, D), jnp.float32),
    )(x1p, x2, s2, c2,
      wa, wb, wc, _pad8(upd_p['b'][0]),
      upd_p['W'][1], _pad8(upd_p['b'][1]),
      upd_p['W'][2], _pad8(upd_p['b'][2]),
      upd_p['W'][3],
      jnp.zeros((8, D), jnp.float32).at[0].set(upd_p['b'][3]))


def _final_body(x2_ref, x1h_ref,
                wp0_ref, bp0_ref, wp1_ref, bp1_ref, wp2_ref, bp2_ref,
                wp3_ref, bp3_ref, wq0_ref, bq0_ref, wq1_ref, bq1_ref,
                out_ref, acc):
    i = pl.program_id(0)
    t = _swish(jnp.dot(x2_ref[...], wp0_ref[...], preferred_element_type=jnp.float32) + bp0_ref[0:1, :])
    t = _swish(jnp.dot(t, wp1_ref[...], preferred_element_type=jnp.float32) + bp1_ref[0:1, :])
    t = _swish(jnp.dot(t, wp2_ref[...], preferred_element_type=jnp.float32) + bp2_ref[0:1, :])
    h = jnp.dot(t, wp3_ref[...], preferred_element_type=jnp.float32) + bp3_ref[0:1, :]

    @pl.when(i == 0)
    def _():
        acc[...] = jnp.zeros_like(acc)

    acc[0:1, :] = acc[0:1, :] + jnp.sum(h, axis=0, keepdims=True)

    @pl.when(i == pl.num_programs(0) - 1)
    def _():
        pooled = acc[...] / N
        c = _swish(jnp.dot(pooled, wq0_ref[...], preferred_element_type=jnp.float32) + bq0_ref[0:1, :])
        coeff = jnp.dot(c, wq1_ref[...], preferred_element_type=jnp.float32) + bq1_ref[0:1, :]
        out_ref[...] = jnp.dot(coeff, x1h_ref[...], preferred_element_type=jnp.float32)


def _final(x2, x1h, pre_p, post_p):
    g = N // NBLK
    cst = lambda shape: pl.BlockSpec(shape, lambda i: tuple(0 for _ in shape))
    return pl.pallas_call(
        _final_body,
        grid=(g,),
        in_specs=[
            pl.BlockSpec((NBLK, D), lambda i: (i, 0)),
            cst((H, H)),
            cst((D, H)), cst((8, H)),
            cst((H, H)), cst((8, H)),
            cst((H, H)), cst((8, H)),
            cst((H, H)), cst((8, H)),
            cst((H, H)), cst((8, H)),
            cst((H, H)), cst((8, H)),
        ],
        out_specs=pl.BlockSpec((8, H), lambda i: (0, 0)),
        out_shape=jax.ShapeDtypeStruct((8, H), jnp.float32),
        scratch_shapes=[pltpu.VMEM((8, H), jnp.float32)],
    )(x2, x1h,
      pre_p['W'][0], _pad8(pre_p['b'][0]),
      pre_p['W'][1], _pad8(pre_p['b'][1]),
      pre_p['W'][2], _pad8(pre_p['b'][2]),
      pre_p['W'][3], _pad8(pre_p['b'][3]),
      post_p['W'][0], _pad8(post_p['b'][0]),
      post_p['W'][1], _pad8(post_p['b'][1]))


NC = 2
NS = 16
NW = NC * NS
EW = NE // NW
GC = 400


_SC_PARAMS = pltpu.CompilerParams(use_tc_tiling_on_sc=False)


def _gather(a, b, dst, src):
    mesh = plsc.VectorSubcoreMesh(core_axis_name="c", subcore_axis_name="s")

    @functools.partial(
        pl.kernel,
        mesh=mesh,
        out_type=[
            jax.ShapeDtypeStruct((NE, H), jnp.float32),
            jax.ShapeDtypeStruct((NE, H), jnp.float32),
        ],
        scratch_types=[
            pltpu.VMEM((2, GC), jnp.int32),
            pltpu.VMEM((2, GC), jnp.int32),
            pltpu.VMEM((2, GC, H), jnp.float32),
            pltpu.VMEM((2, GC, H), jnp.float32),
            pltpu.SemaphoreType.DMA,
            pltpu.SemaphoreType.DMA,
            pltpu.SemaphoreType.DMA,
            pltpu.SemaphoreType.DMA,
        ],
        compiler_params=_SC_PARAMS,
    )
    def k(a_hbm, b_hbm, dst_hbm, src_hbm, ga_hbm, gb_hbm, idxd, idxs,
          rowd, rows, sga, sgb, swa, swb):
        wid = lax.axis_index("s") * NC + lax.axis_index("c")
        base = wid * EW
        nch = EW // GC

        pltpu.sync_copy(dst_hbm.at[pl.ds(base, GC)], idxd.at[0])
        pltpu.sync_copy(src_hbm.at[pl.ds(base, GC)], idxs.at[0])
        pltpu.async_copy(a_hbm.at[idxd.at[0]], rowd.at[0], sga)
        pltpu.async_copy(b_hbm.at[idxs.at[0]], rows.at[0], sgb)

        def body(j, carry):
            b = j % 2
            nb = 1 - b
            e0 = base + j * GC

            @pl.when(j >= 1)
            def _():
                ep = e0 - GC
                pltpu.make_async_copy(
                    rowd.at[nb], ga_hbm.at[pl.ds(ep, GC)], swa).wait()
                pltpu.make_async_copy(
                    rows.at[nb], gb_hbm.at[pl.ds(ep, GC)], swb).wait()

            @pl.when(j + 1 < nch)
            def _():
                e1 = e0 + GC
                pltpu.sync_copy(dst_hbm.at[pl.ds(e1, GC)], idxd.at[nb])
                pltpu.sync_copy(src_hbm.at[pl.ds(e1, GC)], idxs.at[nb])
                pltpu.async_copy(a_hbm.at[idxd.at[nb]], rowd.at[nb], sga)
                pltpu.async_copy(b_hbm.at[idxs.at[nb]], rows.at[nb], sgb)

            pltpu.make_async_copy(a_hbm.at[idxd.at[b]], rowd.at[b], sga).wait()
            pltpu.make_async_copy(b_hbm.at[idxs.at[b]], rows.at[b], sgb).wait()
            pltpu.async_copy(rowd.at[b], ga_hbm.at[pl.ds(e0, GC)], swa)
            pltpu.async_copy(rows.at[b], gb_hbm.at[pl.ds(e0, GC)], swb)
            return carry

        lax.fori_loop(0, nch, body, 0)
        el = base + (nch - 1) * GC
        bl = (nch - 1) % 2
        pltpu.make_async_copy(rowd.at[bl], ga_hbm.at[pl.ds(el, GC)], swa).wait()
        pltpu.make_async_copy(rows.at[bl], gb_hbm.at[pl.ds(el, GC)], swb).wait()

    return k(a, b, dst, src)


def _scatter(m, dst):
    mesh = plsc.VectorSubcoreMesh(core_axis_name="c", subcore_axis_name="s")
    z64 = jnp.zeros((N, H), jnp.float32)
    z16 = jnp.zeros((N, 16), jnp.float32)
    ones = jnp.ones((GC, 16), jnp.float32)
    nrows = N // NS

    @functools.partial(
        pl.kernel,
        mesh=mesh,
        out_type=[
            jax.ShapeDtypeStruct((NC, N, H), jnp.float32),
            jax.ShapeDtypeStruct((NC, N, 16), jnp.float32),
        ],
        scratch_types=[
            pltpu.VMEM((GC,), jnp.int32),
            pltpu.VMEM((GC, H), jnp.float32),
            pltpu.VMEM((GC, 16), jnp.float32),
            pltpu.VMEM_SHARED((N, H), jnp.float32),
            pltpu.VMEM_SHARED((N, 16), jnp.float32),
        ],
        compiler_params=_SC_PARAMS,
    )
    def k(m_hbm, dst_hbm, z64_hbm, z16_hbm, ones_hbm, s_hbm, c_hbm,
          idx, rows, onev, acc, accc):
        cid = lax.axis_index("c")
        sid = lax.axis_index("s")
        wid = sid * NC + cid
        base = wid * EW
        pltpu.sync_copy(ones_hbm, onev)

        @pl.when(sid == 0)
        def _():
            pltpu.sync_copy(z64_hbm, acc)
            pltpu.sync_copy(z16_hbm, accc)

        plsc.subcore_barrier()

        def body(j, carry):
            e0 = base + j * GC
            pltpu.sync_copy(dst_hbm.at[pl.ds(e0, GC)], idx)
            pltpu.sync_copy(m_hbm.at[pl.ds(e0, GC)], rows)
            pltpu.sync_copy(rows, acc.at[idx], add=True)
            pltpu.sync_copy(onev, accc.at[idx], add=True)
            return carry

        lax.fori_loop(0, EW // GC, body, 0)
        plsc.subcore_barrier()
        r0 = sid * nrows
        pltpu.sync_copy(acc.at[pl.ds(r0, nrows)], s_hbm.at[cid, pl.ds(r0, nrows)])
        pltpu.sync_copy(accc.at[pl.ds(r0, nrows)], c_hbm.at[cid, pl.ds(r0, nrows)])

    return k(m, dst, z64, z16, ones)


def kernel(node_feature, vectors, params, edge_index):
    x0 = node_feature[0]
    src = edge_index[0, 0]
    dst = edge_index[0, 1]
    x1 = x0[:, :V]
    x1p = jnp.pad(x1, ((0, 0), (0, H - V)))
    x1h = jnp.pad(x1[:H], ((0, 0), (0, H - V)))

    x2 = x0
    for lp in params['gnn']:
        msg = lp['msg']
        w0 = msg['W'][0]
        w0a = w0[:D]
        w0b = jnp.zeros((D, H), jnp.float32).at[:V].set(w0[D:])
        a, b = _proj(x2, w0a, w0b)
        ga, gb = _gather(a, b, dst, src)
        m = _msgnet(ga.reshape(NE2, D2), gb.reshape(NE2, D2), msg)
        s2, c2 = _scatter(m.reshape(NE, H), dst)
        x2 = _upd(x1p, x2, s2, c2, lp['upd'])

    out = _final(x2, x1h, params['pre'], params['post'])
    return out[0, :V]


# speedup vs baseline: 2.1653x; 1.0710x over previous
import functools

import jax
import jax.numpy as jnp
from jax import lax
from jax.experimental import pallas as pl
from jax.experimental.pallas import tpu as pltpu
from jax.experimental.pallas import tpu_sc as plsc

N = 10000
NE = 320000
D = 128
V = 42
H = 64
EBLK = 8000
NBLK = 2000
EPS = 1e-5


def _swish(x):
    return x * (0.5 * jnp.tanh(0.5 * x) + 0.5)


def _pad8(b):
    return jnp.zeros((8, H), jnp.float32).at[0].set(b)


def _proj_body(x_ref, wa_ref, wb_ref, a_ref, b_ref):
    x = x_ref[...]
    a_ref[...] = jnp.dot(x, wa_ref[...], preferred_element_type=jnp.float32)
    b_ref[...] = jnp.dot(x, wb_ref[...], preferred_element_type=jnp.float32)


def _proj(x2, w0a, w0b):
    g = N // NBLK
    return pl.pallas_call(
        _proj_body,
        grid=(g,),
        in_specs=[
            pl.BlockSpec((NBLK, D), lambda i: (i, 0)),
            pl.BlockSpec((D, H), lambda i: (0, 0)),
            pl.BlockSpec((D, H), lambda i: (0, 0)),
        ],
        out_specs=[
            pl.BlockSpec((NBLK, H), lambda i: (i, 0)),
            pl.BlockSpec((NBLK, H), lambda i: (i, 0)),
        ],
        out_shape=[
            jax.ShapeDtypeStruct((N, H), jnp.float32),
            jax.ShapeDtypeStruct((N, H), jnp.float32),
        ],
    )(x2, w0a, w0b)


NE2 = NE // 2
EBLK2 = EBLK // 2
D2 = 2 * H


def _msg_body(ga_ref, gb_ref, b0_ref, gbe_ref, w1_ref, w2_ref, w3_ref,
              b123_ref, psum_ref, x_ref, m_ref, acc, nrm, wsel):
    p = pl.program_id(0)
    i = pl.program_id(1)

    @pl.when((p == 0) & (i == 0))
    def _():
        acc[...] = jnp.zeros_like(acc)

    @pl.when((p >= 1) & (i == 0))
    def _():
        mu = jnp.dot(acc[0:1, :], psum_ref[...],
                     preferred_element_type=jnp.float32) * (1.0 / NE)
        msq = jnp.dot(acc[1:2, :], psum_ref[...],
                      preferred_element_type=jnp.float32) * (1.0 / NE)
        var = msq - mu * mu
        g = jnp.where(p == 1, gbe_ref[0:1, :],
                      jnp.where(p == 2, gbe_ref[2:3, :], gbe_ref[4:5, :]))
        be = jnp.where(p == 1, gbe_ref[1:2, :],
                       jnp.where(p == 2, gbe_ref[3:4, :], gbe_ref[5:6, :]))
        s = g * lax.rsqrt(var + EPS)
        nrm[0:1, :] = s
        nrm[1:2, :] = be - mu * s
        nrm[2:3, :] = jnp.where(p == 1, b123_ref[0:1, :],
                                jnp.where(p == 2, b123_ref[1:2, :],
                                          b123_ref[2:3, :]))
        wsel[...] = jnp.where(p == 1, w1_ref[...],
                              jnp.where(p == 2, w2_ref[...], w3_ref[...]))
        acc[...] = jnp.zeros_like(acc)

    @pl.when(p == 0)
    def _():
        h = _swish(ga_ref[...] + gb_ref[...] + b0_ref[0:1, :])
        x_ref[...] = h
        acc[0:1, :] = acc[0:1, :] + jnp.sum(h, axis=0, keepdims=True)
        acc[1:2, :] = acc[1:2, :] + jnp.sum(h * h, axis=0, keepdims=True)

    @pl.when(p >= 1)
    def _():
        xn = ga_ref[...] * nrm[0:1, :] + nrm[1:2, :]
        y = _swish(
            jnp.dot(xn, wsel[...], preferred_element_type=jnp.float32)
            + nrm[2:3, :]
        )

        @pl.when(p < 3)
        def _():
            x_ref[...] = y
            acc[0:1, :] = acc[0:1, :] + jnp.sum(y, axis=0, keepdims=True)
            acc[1:2, :] = acc[1:2, :] + jnp.sum(y * y, axis=0, keepdims=True)

        @pl.when(p == 3)
        def _():
            m_ref[...] = y


def _dup(v):
    return jnp.concatenate([v, v])


def _pad8d(rows):
    out = jnp.zeros((8, D2), jnp.float32)
    for r, v in enumerate(rows):
        out = out.at[r].set(_dup(v))
    return out


def _bdiag(w):
    return (jnp.zeros((D2, D2), jnp.float32)
            .at[:H, :H].set(w).at[H:, H:].set(w))


def _msgnet(ga, gb, msg):
    ge = NE2 // EBLK2
    b0p = _pad8d([msg['b'][0]])
    gbe = _pad8d([msg['g'][0], msg['be'][0], msg['g'][1], msg['be'][1],
                  msg['g'][2], msg['be'][2]])
    b123 = _pad8d([msg['b'][1], msg['b'][2], msg['b'][3]])
    eye = jnp.eye(H, dtype=jnp.float32)
    psum = jnp.block([[eye, eye], [eye, eye]])
    cst = lambda shape: pl.BlockSpec(shape, lambda p, i: tuple(0 for _ in shape))
    _, m = pl.pallas_call(
        _msg_body,
        grid=(4, ge),
        in_specs=[
            pl.BlockSpec((EBLK2, D2), lambda p, i: (i, 0)),
            pl.BlockSpec((EBLK2, D2), lambda p, i: (jnp.where(p == 0, i, 0), 0)),
            cst((8, D2)), cst((8, D2)),
            cst((D2, D2)), cst((D2, D2)), cst((D2, D2)), cst((8, D2)),
            cst((D2, D2)),
        ],
        out_specs=[
            pl.BlockSpec((EBLK2, D2), lambda p, i: (jnp.where(p < 3, i, 0), 0)),
            pl.BlockSpec((EBLK2, D2), lambda p, i: (jnp.where(p == 3, i, 0), 0)),
        ],
        out_shape=[
            jax.ShapeDtypeStruct((NE2, D2), jnp.float32),
            jax.ShapeDtypeStruct((NE2, D2), jnp.float32),
        ],
        scratch_shapes=[
            pltpu.VMEM((8, D2), jnp.float32),
            pltpu.VMEM((8, D2), jnp.float32),
            pltpu.VMEM((D2, D2), jnp.float32),
        ],
        input_output_aliases={0: 0},
    )(ga, gb, b0p, gbe, _bdiag(msg['W'][1]), _bdiag(msg['W'][2]),
      _bdiag(msg['W'][3]), b123, psum)
    return m


def _upd_body(x1_ref, x2_ref, s2_ref, c2_ref,
              wa_ref, wb_ref, wc_ref, b0_ref, w1_ref, b1_ref,
              w2_ref, b2_ref, w3_ref, b3_ref, xo_ref):
    cnt = c2_ref[0, :, 0:1] + c2_ref[1, :, 0:1]
    cnt = jnp.maximum(cnt, 1.0)
    agg = (s2_ref[0] + s2_ref[1]) / cnt
    t = _swish(
        jnp.dot(x1_ref[...], wa_ref[...], preferred_element_type=jnp.float32)
        + jnp.dot(x2_ref[...], wb_ref[...], preferred_element_type=jnp.float32)
        + jnp.dot(agg, wc_ref[...], preferred_element_type=jnp.float32)
        + b0_ref[0:1, :]
    )
    t = _swish(jnp.dot(t, w1_ref[...], preferred_element_type=jnp.float32) + b1_ref[0:1, :])
    t = _swish(jnp.dot(t, w2_ref[...], preferred_element_type=jnp.float32) + b2_ref[0:1, :])
    t = _swish(jnp.dot(t, w3_ref[...], preferred_element_type=jnp.float32) + b3_ref[0:1, :])
    xo_ref[...] = x2_ref[...] + t


def _upd(x1p, x2, s2, c2, upd_p):
    wu0 = upd_p['W'][0]
    wa = jnp.zeros((H, H), jnp.float32).at[:V].set(wu0[:V])
    wb = wu0[V:V + D]
    wc = wu0[V + D:]
    g = N // NBLK
    cst = lambda shape: pl.BlockSpec(shape, lambda i: tuple(0 for _ in shape))
    return pl.pallas_call(
        _upd_body,
        grid=(g,),
        in_specs=[
            pl.BlockSpec((NBLK, H), lambda i: (i, 0)),
            pl.BlockSpec((NBLK, D), lambda i: (i, 0)),
            pl.BlockSpec((2, NBLK, H), lambda i: (0, i, 0)),
            pl.BlockSpec((2, NBLK, 16), lambda i: (0, i, 0)),
            cst((H, H)), cst((D, H)), cst((H, H)), cst((8, H)),
            cst((H, H)), cst((8, H)),
            cst((H, H)), cst((8, H)),
            cst((H, D)), cst((8, D)),
        ],
        out_specs=pl.BlockSpec((NBLK, D), lambda i: (i, 0)),
        out_shape=jax.ShapeDtypeStruct((N, D), jnp.float32),
    )(x1p, x2, s2, c2,
      wa, wb, wc, _pad8(upd_p['b'][0]),
      upd_p['W'][1], _pad8(upd_p['b'][1]),
      upd_p['W'][2], _pad8(upd_p['b'][2]),
      upd_p['W'][3],
      jnp.zeros((8, D), jnp.float32).at[0].set(upd_p['b'][3]))


def _final_body(x2_ref, x1h_ref,
                wp0_ref, bp0_ref, wp1_ref, bp1_ref, wp2_ref, bp2_ref,
                wp3_ref, bp3_ref, wq0_ref, bq0_ref, wq1_ref, bq1_ref,
                out_ref, acc):
    i = pl.program_id(0)
    t = _swish(jnp.dot(x2_ref[...], wp0_ref[...], preferred_element_type=jnp.float32) + bp0_ref[0:1, :])
    t = _swish(jnp.dot(t, wp1_ref[...], preferred_element_type=jnp.float32) + bp1_ref[0:1, :])
    t = _swish(jnp.dot(t, wp2_ref[...], preferred_element_type=jnp.float32) + bp2_ref[0:1, :])
    h = jnp.dot(t, wp3_ref[...], preferred_element_type=jnp.float32) + bp3_ref[0:1, :]

    @pl.when(i == 0)
    def _():
        acc[...] = jnp.zeros_like(acc)

    acc[0:1, :] = acc[0:1, :] + jnp.sum(h, axis=0, keepdims=True)

    @pl.when(i == pl.num_programs(0) - 1)
    def _():
        pooled = acc[...] / N
        c = _swish(jnp.dot(pooled, wq0_ref[...], preferred_element_type=jnp.float32) + bq0_ref[0:1, :])
        coeff = jnp.dot(c, wq1_ref[...], preferred_element_type=jnp.float32) + bq1_ref[0:1, :]
        out_ref[...] = jnp.dot(coeff, x1h_ref[...], preferred_element_type=jnp.float32)


def _final(x2, x1h, pre_p, post_p):
    g = N // NBLK
    cst = lambda shape: pl.BlockSpec(shape, lambda i: tuple(0 for _ in shape))
    return pl.pallas_call(
        _final_body,
        grid=(g,),
        in_specs=[
            pl.BlockSpec((NBLK, D), lambda i: (i, 0)),
            cst((H, H)),
            cst((D, H)), cst((8, H)),
            cst((H, H)), cst((8, H)),
            cst((H, H)), cst((8, H)),
            cst((H, H)), cst((8, H)),
            cst((H, H)), cst((8, H)),
            cst((H, H)), cst((8, H)),
        ],
        out_specs=pl.BlockSpec((8, H), lambda i: (0, 0)),
        out_shape=jax.ShapeDtypeStruct((8, H), jnp.float32),
        scratch_shapes=[pltpu.VMEM((8, H), jnp.float32)],
    )(x2, x1h,
      pre_p['W'][0], _pad8(pre_p['b'][0]),
      pre_p['W'][1], _pad8(pre_p['b'][1]),
      pre_p['W'][2], _pad8(pre_p['b'][2]),
      pre_p['W'][3], _pad8(pre_p['b'][3]),
      post_p['W'][0], _pad8(post_p['b'][0]),
      post_p['W'][1], _pad8(post_p['b'][1]))


NC = 2
NS = 16
NW = NC * NS
EW = NE // NW
GC = 400


_SC_PARAMS = pltpu.CompilerParams(use_tc_tiling_on_sc=False)


def _gather(a, b, dst, src):
    mesh = plsc.VectorSubcoreMesh(core_axis_name="c", subcore_axis_name="s")

    @functools.partial(
        pl.kernel,
        mesh=mesh,
        out_type=[
            jax.ShapeDtypeStruct((NE, H), jnp.float32),
            jax.ShapeDtypeStruct((NE, H), jnp.float32),
        ],
        scratch_types=[
            pltpu.VMEM((2, GC), jnp.int32),
            pltpu.VMEM((2, GC), jnp.int32),
            pltpu.VMEM((2, GC, H), jnp.float32),
            pltpu.VMEM((2, GC, H), jnp.float32),
            pltpu.SemaphoreType.DMA,
            pltpu.SemaphoreType.DMA,
            pltpu.SemaphoreType.DMA,
            pltpu.SemaphoreType.DMA,
        ],
        compiler_params=_SC_PARAMS,
    )
    def k(a_hbm, b_hbm, dst_hbm, src_hbm, ga_hbm, gb_hbm, idxd, idxs,
          rowd, rows, sga, sgb, swa, swb):
        wid = lax.axis_index("s") * NC + lax.axis_index("c")
        base = wid * EW
        nch = EW // GC

        pltpu.sync_copy(dst_hbm.at[pl.ds(base, GC)], idxd.at[0])
        pltpu.sync_copy(src_hbm.at[pl.ds(base, GC)], idxs.at[0])
        pltpu.async_copy(a_hbm.at[idxd.at[0]], rowd.at[0], sga)
        pltpu.async_copy(b_hbm.at[idxs.at[0]], rows.at[0], sgb)

        def body(j, carry):
            b = j % 2
            nb = 1 - b
            e0 = base + j * GC

            @pl.when(j >= 1)
            def _():
                ep = e0 - GC
                pltpu.make_async_copy(
                    rowd.at[nb], ga_hbm.at[pl.ds(ep, GC)], swa).wait()
                pltpu.make_async_copy(
                    rows.at[nb], gb_hbm.at[pl.ds(ep, GC)], swb).wait()

            @pl.when(j + 1 < nch)
            def _():
                e1 = e0 + GC
                pltpu.sync_copy(dst_hbm.at[pl.ds(e1, GC)], idxd.at[nb])
                pltpu.sync_copy(src_hbm.at[pl.ds(e1, GC)], idxs.at[nb])
                pltpu.async_copy(a_hbm.at[idxd.at[nb]], rowd.at[nb], sga)
                pltpu.async_copy(b_hbm.at[idxs.at[nb]], rows.at[nb], sgb)

            pltpu.make_async_copy(a_hbm.at[idxd.at[b]], rowd.at[b], sga).wait()
            pltpu.make_async_copy(b_hbm.at[idxs.at[b]], rows.at[b], sgb).wait()
            pltpu.async_copy(rowd.at[b], ga_hbm.at[pl.ds(e0, GC)], swa)
            pltpu.async_copy(rows.at[b], gb_hbm.at[pl.ds(e0, GC)], swb)
            return carry

        lax.fori_loop(0, nch, body, 0)
        el = base + (nch - 1) * GC
        bl = (nch - 1) % 2
        pltpu.make_async_copy(rowd.at[bl], ga_hbm.at[pl.ds(el, GC)], swa).wait()
        pltpu.make_async_copy(rows.at[bl], gb_hbm.at[pl.ds(el, GC)], swb).wait()

    return k(a, b, dst, src)


def _scatter(m, dst, with_cnt):
    mesh = plsc.VectorSubcoreMesh(core_axis_name="c", subcore_axis_name="s")
    z64 = jnp.zeros((N, H), jnp.float32)
    z16 = jnp.zeros((N, 16), jnp.float32)
    ones = jnp.ones((GC, 16), jnp.float32)
    nrows = N // NS
    out_type = [jax.ShapeDtypeStruct((NC, N, H), jnp.float32)]
    if with_cnt:
        out_type.append(jax.ShapeDtypeStruct((NC, N, 16), jnp.float32))

    @functools.partial(
        pl.kernel,
        mesh=mesh,
        out_type=out_type,
        scratch_types=[
            pltpu.VMEM((2, GC), jnp.int32),
            pltpu.VMEM((2, GC, H), jnp.float32),
            pltpu.VMEM((GC, 16), jnp.float32),
            pltpu.VMEM_SHARED((N, H), jnp.float32),
            pltpu.VMEM_SHARED((N, 16), jnp.float32),
            pltpu.SemaphoreType.DMA,
            pltpu.SemaphoreType.DMA,
        ],
        compiler_params=_SC_PARAMS,
    )
    def k(m_hbm, dst_hbm, z64_hbm, z16_hbm, ones_hbm, *outs_and_scratch):
        if with_cnt:
            s_hbm, c_hbm = outs_and_scratch[:2]
            idx, rows, onev, acc, accc, si, sm = outs_and_scratch[2:]
        else:
            s_hbm = outs_and_scratch[0]
            c_hbm = None
            idx, rows, onev, acc, accc, si, sm = outs_and_scratch[1:]
        cid = lax.axis_index("c")
        sid = lax.axis_index("s")
        wid = sid * NC + cid
        base = wid * EW
        nch = EW // GC
        if with_cnt:
            pltpu.sync_copy(ones_hbm, onev)

        @pl.when(sid == 0)
        def _():
            pltpu.sync_copy(z64_hbm, acc)
            if with_cnt:
                pltpu.sync_copy(z16_hbm, accc)

        pltpu.async_copy(dst_hbm.at[pl.ds(base, GC)], idx.at[0], si)
        pltpu.async_copy(m_hbm.at[pl.ds(base, GC)], rows.at[0], sm)
        plsc.subcore_barrier()

        def body(j, carry):
            b = j % 2
            nb = 1 - b
            e0 = base + j * GC
            pltpu.make_async_copy(
                dst_hbm.at[pl.ds(e0, GC)], idx.at[b], si).wait()
            pltpu.make_async_copy(
                m_hbm.at[pl.ds(e0, GC)], rows.at[b], sm).wait()

            @pl.when(j + 1 < nch)
            def _():
                e1 = e0 + GC
                pltpu.async_copy(dst_hbm.at[pl.ds(e1, GC)], idx.at[nb], si)
                pltpu.async_copy(m_hbm.at[pl.ds(e1, GC)], rows.at[nb], sm)

            pltpu.sync_copy(rows.at[b], acc.at[idx.at[b]], add=True)
            if with_cnt:
                pltpu.sync_copy(onev, accc.at[idx.at[b]], add=True)
            return carry

        lax.fori_loop(0, nch, body, 0)
        plsc.subcore_barrier()
        r0 = sid * nrows
        pltpu.sync_copy(acc.at[pl.ds(r0, nrows)], s_hbm.at[cid, pl.ds(r0, nrows)])
        if with_cnt:
            pltpu.sync_copy(accc.at[pl.ds(r0, nrows)],
                            c_hbm.at[cid, pl.ds(r0, nrows)])

    return k(m, dst, z64, z16, ones)


def kernel(node_feature, vectors, params, edge_index):
    x0 = node_feature[0]
    src = edge_index[0, 0]
    dst = edge_index[0, 1]
    x1 = x0[:, :V]
    x1p = jnp.pad(x1, ((0, 0), (0, H - V)))
    x1h = jnp.pad(x1[:H], ((0, 0), (0, H - V)))

    x2 = x0
    c2 = None
    for li, lp in enumerate(params['gnn']):
        msg = lp['msg']
        w0 = msg['W'][0]
        w0a = w0[:D]
        w0b = jnp.zeros((D, H), jnp.float32).at[:V].set(w0[D:])
        a, b = _proj(x2, w0a, w0b)
        ga, gb = _gather(a, b, dst, src)
        m = _msgnet(ga.reshape(NE2, D2), gb.reshape(NE2, D2), msg)
        if li == 0:
            s2, c2 = _scatter(m.reshape(NE, H), dst, with_cnt=True)
        else:
            (s2,) = _scatter(m.reshape(NE, H), dst, with_cnt=False)
        x2 = _upd(x1p, x2, s2, c2, lp['upd'])

    out = _final(x2, x1h, params['pre'], params['post'])
    return out[0, :V]


# speedup vs baseline: 2.3322x; 1.0770x over previous
import functools

import jax
import jax.numpy as jnp
from jax import lax
from jax.experimental import pallas as pl
from jax.experimental.pallas import tpu as pltpu
from jax.experimental.pallas import tpu_sc as plsc

N = 10000
NE = 320000
D = 128
V = 42
H = 64
EBLK = 8000
NBLK = 2000
EPS = 1e-5


def _swish(x):
    return x * (0.5 * jnp.tanh(0.5 * x) + 0.5)


def _pad8(b):
    return jnp.zeros((8, H), jnp.float32).at[0].set(b)


def _proj_body(x_ref, wa_ref, wb_ref, a_ref, b_ref):
    x = x_ref[...]
    a_ref[...] = jnp.dot(x, wa_ref[...], preferred_element_type=jnp.float32)
    b_ref[...] = jnp.dot(x, wb_ref[...], preferred_element_type=jnp.float32)


def _proj(x2, w0a, w0b):
    g = N // NBLK
    return pl.pallas_call(
        _proj_body,
        grid=(g,),
        in_specs=[
            pl.BlockSpec((NBLK, D), lambda i: (i, 0)),
            pl.BlockSpec((D, H), lambda i: (0, 0)),
            pl.BlockSpec((D, H), lambda i: (0, 0)),
        ],
        out_specs=[
            pl.BlockSpec((NBLK, H), lambda i: (i, 0)),
            pl.BlockSpec((NBLK, H), lambda i: (i, 0)),
        ],
        out_shape=[
            jax.ShapeDtypeStruct((N, H), jnp.float32),
            jax.ShapeDtypeStruct((N, H), jnp.float32),
        ],
    )(x2, w0a, w0b)


NE2 = NE // 2
EBLK2 = EBLK // 2
D2 = 2 * H


def _msg0_body(ga_ref, gb_ref, b0_ref, h_ref, st_ref, acc):
    i = pl.program_id(0)
    h = _swish(ga_ref[...] + gb_ref[...] + b0_ref[0:1, :])
    hb = h.astype(jnp.bfloat16)
    h_ref[...] = hb
    hf = hb.astype(jnp.float32)

    @pl.when(i == 0)
    def _():
        acc[...] = jnp.zeros_like(acc)

    acc[0:1, :] = acc[0:1, :] + jnp.sum(hf, axis=0, keepdims=True)
    acc[1:2, :] = acc[1:2, :] + jnp.sum(hf * hf, axis=0, keepdims=True)

    @pl.when(i == pl.num_programs(0) - 1)
    def _():
        st_ref[...] = acc[...]


def _msg123_body(x_in_ref, st_ref, gbe_ref, w1_ref, w2_ref, w3_ref,
                 b123_ref, psum_ref, x_ref, m_ref, acc, nrm, wsel):
    p = pl.program_id(0)
    i = pl.program_id(1)

    @pl.when(i == 0)
    def _():
        s0 = jnp.where(p == 0, st_ref[0:1, :], acc[0:1, :])
        s1 = jnp.where(p == 0, st_ref[1:2, :], acc[1:2, :])
        mu = jnp.dot(s0, psum_ref[...],
                     preferred_element_type=jnp.float32) * (1.0 / NE)
        msq = jnp.dot(s1, psum_ref[...],
                      preferred_element_type=jnp.float32) * (1.0 / NE)
        var = msq - mu * mu
        g = jnp.where(p == 0, gbe_ref[0:1, :],
                      jnp.where(p == 1, gbe_ref[2:3, :], gbe_ref[4:5, :]))
        be = jnp.where(p == 0, gbe_ref[1:2, :],
                       jnp.where(p == 1, gbe_ref[3:4, :], gbe_ref[5:6, :]))
        s = g * lax.rsqrt(var + EPS)
        nrm[0:1, :] = s
        nrm[1:2, :] = be - mu * s
        nrm[2:3, :] = jnp.where(p == 0, b123_ref[0:1, :],
                                jnp.where(p == 1, b123_ref[1:2, :],
                                          b123_ref[2:3, :]))
        wsel[...] = jnp.where(p == 0, w1_ref[...],
                              jnp.where(p == 1, w2_ref[...], w3_ref[...]))
        acc[...] = jnp.zeros_like(acc)

    xn = x_in_ref[...].astype(jnp.float32) * nrm[0:1, :] + nrm[1:2, :]
    y = _swish(
        jnp.dot(xn, wsel[...], preferred_element_type=jnp.float32)
        + nrm[2:3, :]
    )

    @pl.when(p < 2)
    def _():
        yb = y.astype(jnp.bfloat16)
        x_ref[...] = yb
        yf = yb.astype(jnp.float32)
        acc[0:1, :] = acc[0:1, :] + jnp.sum(yf, axis=0, keepdims=True)
        acc[1:2, :] = acc[1:2, :] + jnp.sum(yf * yf, axis=0, keepdims=True)

    @pl.when(p == 2)
    def _():
        m_ref[...] = y


def _dup(v):
    return jnp.concatenate([v, v])


def _pad8d(rows):
    out = jnp.zeros((8, D2), jnp.float32)
    for r, v in enumerate(rows):
        out = out.at[r].set(_dup(v))
    return out


def _bdiag(w):
    return (jnp.zeros((D2, D2), jnp.float32)
            .at[:H, :H].set(w).at[H:, H:].set(w))


def _msgnet(ga, gb, msg):
    ge = NE2 // EBLK2
    b0p = _pad8d([msg['b'][0]])
    gbe = _pad8d([msg['g'][0], msg['be'][0], msg['g'][1], msg['be'][1],
                  msg['g'][2], msg['be'][2]])
    b123 = _pad8d([msg['b'][1], msg['b'][2], msg['b'][3]])
    eye = jnp.eye(H, dtype=jnp.float32)
    psum = jnp.block([[eye, eye], [eye, eye]])
    h1, st1 = pl.pallas_call(
        _msg0_body,
        grid=(ge,),
        in_specs=[
            pl.BlockSpec((EBLK2, D2), lambda i: (i, 0)),
            pl.BlockSpec((EBLK2, D2), lambda i: (i, 0)),
            pl.BlockSpec((8, D2), lambda i: (0, 0)),
        ],
        out_specs=[
            pl.BlockSpec((EBLK2, D2), lambda i: (i, 0)),
            pl.BlockSpec((8, D2), lambda i: (0, 0)),
        ],
        out_shape=[
            jax.ShapeDtypeStruct((NE2, D2), jnp.bfloat16),
            jax.ShapeDtypeStruct((8, D2), jnp.float32),
        ],
        scratch_shapes=[pltpu.VMEM((8, D2), jnp.float32)],
    )(ga, gb, b0p)

    cst = lambda shape: pl.BlockSpec(shape, lambda p, i: tuple(0 for _ in shape))
    _, m = pl.pallas_call(
        _msg123_body,
        grid=(3, ge),
        in_specs=[
            pl.BlockSpec((EBLK2, D2), lambda p, i: (i, 0)),
            cst((8, D2)), cst((8, D2)),
            cst((D2, D2)), cst((D2, D2)), cst((D2, D2)), cst((8, D2)),
            cst((D2, D2)),
        ],
        out_specs=[
            pl.BlockSpec((EBLK2, D2), lambda p, i: (jnp.where(p < 2, i, 0), 0)),
            pl.BlockSpec((EBLK2, D2), lambda p, i: (jnp.where(p == 2, i, 0), 0)),
        ],
        out_shape=[
            jax.ShapeDtypeStruct((NE2, D2), jnp.bfloat16),
            jax.ShapeDtypeStruct((NE2, D2), jnp.float32),
        ],
        scratch_shapes=[
            pltpu.VMEM((8, D2), jnp.float32),
            pltpu.VMEM((8, D2), jnp.float32),
            pltpu.VMEM((D2, D2), jnp.float32),
        ],
        input_output_aliases={0: 0},
    )(h1, st1, gbe, _bdiag(msg['W'][1]), _bdiag(msg['W'][2]),
      _bdiag(msg['W'][3]), b123, psum)
    return m


def _upd_body(x1_ref, x2_ref, s2_ref, c2_ref,
              wa_ref, wb_ref, wc_ref, b0_ref, w1_ref, b1_ref,
              w2_ref, b2_ref, w3_ref, b3_ref, xo_ref):
    cnt = c2_ref[0, :, 0:1] + c2_ref[1, :, 0:1]
    cnt = jnp.maximum(cnt, 1.0)
    agg = (s2_ref[0] + s2_ref[1]) / cnt
    t = _swish(
        jnp.dot(x1_ref[...], wa_ref[...], preferred_element_type=jnp.float32)
        + jnp.dot(x2_ref[...], wb_ref[...], preferred_element_type=jnp.float32)
        + jnp.dot(agg, wc_ref[...], preferred_element_type=jnp.float32)
        + b0_ref[0:1, :]
    )
    t = _swish(jnp.dot(t, w1_ref[...], preferred_element_type=jnp.float32) + b1_ref[0:1, :])
    t = _swish(jnp.dot(t, w2_ref[...], preferred_element_type=jnp.float32) + b2_ref[0:1, :])
    t = _swish(jnp.dot(t, w3_ref[...], preferred_element_type=jnp.float32) + b3_ref[0:1, :])
    xo_ref[...] = x2_ref[...] + t


def _upd(x1p, x2, s2, c2, upd_p):
    wu0 = upd_p['W'][0]
    wa = jnp.zeros((H, H), jnp.float32).at[:V].set(wu0[:V])
    wb = wu0[V:V + D]
    wc = wu0[V + D:]
    g = N // NBLK
    cst = lambda shape: pl.BlockSpec(shape, lambda i: tuple(0 for _ in shape))
    return pl.pallas_call(
        _upd_body,
        grid=(g,),
        in_specs=[
            pl.BlockSpec((NBLK, H), lambda i: (i, 0)),
            pl.BlockSpec((NBLK, D), lambda i: (i, 0)),
            pl.BlockSpec((2, NBLK, H), lambda i: (0, i, 0)),
            pl.BlockSpec((2, NBLK, 16), lambda i: (0, i, 0)),
            cst((H, H)), cst((D, H)), cst((H, H)), cst((8, H)),
            cst((H, H)), cst((8, H)),
            cst((H, H)), cst((8, H)),
            cst((H, D)), cst((8, D)),
        ],
        out_specs=pl.BlockSpec((NBLK, D), lambda i: (i, 0)),
        out_shape=jax.ShapeDtypeStruct((N, D), jnp.float32),
    )(x1p, x2, s2, c2,
      wa, wb, wc, _pad8(upd_p['b'][0]),
      upd_p['W'][1], _pad8(upd_p['b'][1]),
      upd_p['W'][2], _pad8(upd_p['b'][2]),
      upd_p['W'][3],
      jnp.zeros((8, D), jnp.float32).at[0].set(upd_p['b'][3]))


def _final_body(x2_ref, x1h_ref,
                wp0_ref, bp0_ref, wp1_ref, bp1_ref, wp2_ref, bp2_ref,
                wp3_ref, bp3_ref, wq0_ref, bq0_ref, wq1_ref, bq1_ref,
                out_ref, acc):
    i = pl.program_id(0)
    t = _swish(jnp.dot(x2_ref[...], wp0_ref[...], preferred_element_type=jnp.float32) + bp0_ref[0:1, :])
    t = _swish(jnp.dot(t, wp1_ref[...], preferred_element_type=jnp.float32) + bp1_ref[0:1, :])
    t = _swish(jnp.dot(t, wp2_ref[...], preferred_element_type=jnp.float32) + bp2_ref[0:1, :])
    h = jnp.dot(t, wp3_ref[...], preferred_element_type=jnp.float32) + bp3_ref[0:1, :]

    @pl.when(i == 0)
    def _():
        acc[...] = jnp.zeros_like(acc)

    acc[0:1, :] = acc[0:1, :] + jnp.sum(h, axis=0, keepdims=True)

    @pl.when(i == pl.num_programs(0) - 1)
    def _():
        pooled = acc[...] / N
        c = _swish(jnp.dot(pooled, wq0_ref[...], preferred_element_type=jnp.float32) + bq0_ref[0:1, :])
        coeff = jnp.dot(c, wq1_ref[...], preferred_element_type=jnp.float32) + bq1_ref[0:1, :]
        out_ref[...] = jnp.dot(coeff, x1h_ref[...], preferred_element_type=jnp.float32)


def _final(x2, x1h, pre_p, post_p):
    g = N // NBLK
    cst = lambda shape: pl.BlockSpec(shape, lambda i: tuple(0 for _ in shape))
    return pl.pallas_call(
        _final_body,
        grid=(g,),
        in_specs=[
            pl.BlockSpec((NBLK, D), lambda i: (i, 0)),
            cst((H, H)),
            cst((D, H)), cst((8, H)),
            cst((H, H)), cst((8, H)),
            cst((H, H)), cst((8, H)),
            cst((H, H)), cst((8, H)),
            cst((H, H)), cst((8, H)),
            cst((H, H)), cst((8, H)),
        ],
        out_specs=pl.BlockSpec((8, H), lambda i: (0, 0)),
        out_shape=jax.ShapeDtypeStruct((8, H), jnp.float32),
        scratch_shapes=[pltpu.VMEM((8, H), jnp.float32)],
    )(x2, x1h,
      pre_p['W'][0], _pad8(pre_p['b'][0]),
      pre_p['W'][1], _pad8(pre_p['b'][1]),
      pre_p['W'][2], _pad8(pre_p['b'][2]),
      pre_p['W'][3], _pad8(pre_p['b'][3]),
      post_p['W'][0], _pad8(post_p['b'][0]),
      post_p['W'][1], _pad8(post_p['b'][1]))


NC = 2
NS = 16
NW = NC * NS
EW = NE // NW
GC = 400


_SC_PARAMS = pltpu.CompilerParams(use_tc_tiling_on_sc=False)


def _gather(a, b, dst, src):
    mesh = plsc.VectorSubcoreMesh(core_axis_name="c", subcore_axis_name="s")

    @functools.partial(
        pl.kernel,
        mesh=mesh,
        out_type=[
            jax.ShapeDtypeStruct((NE, H), jnp.float32),
            jax.ShapeDtypeStruct((NE, H), jnp.float32),
        ],
        scratch_types=[
            pltpu.VMEM((2, GC), jnp.int32),
            pltpu.VMEM((2, GC), jnp.int32),
            pltpu.VMEM((2, GC, H), jnp.float32),
            pltpu.VMEM((2, GC, H), jnp.float32),
            pltpu.SemaphoreType.DMA,
            pltpu.SemaphoreType.DMA,
            pltpu.SemaphoreType.DMA,
            pltpu.SemaphoreType.DMA,
        ],
        compiler_params=_SC_PARAMS,
    )
    def k(a_hbm, b_hbm, dst_hbm, src_hbm, ga_hbm, gb_hbm, idxd, idxs,
          rowd, rows, sga, sgb, swa, swb):
        wid = lax.axis_index("s") * NC + lax.axis_index("c")
        base = wid * EW
        nch = EW // GC

        pltpu.sync_copy(dst_hbm.at[pl.ds(base, GC)], idxd.at[0])
        pltpu.sync_copy(src_hbm.at[pl.ds(base, GC)], idxs.at[0])
        pltpu.async_copy(a_hbm.at[idxd.at[0]], rowd.at[0], sga)
        pltpu.async_copy(b_hbm.at[idxs.at[0]], rows.at[0], sgb)

        def body(j, carry):
            b = j % 2
            nb = 1 - b
            e0 = base + j * GC

            @pl.when(j >= 1)
            def _():
                ep = e0 - GC
                pltpu.make_async_copy(
                    rowd.at[nb], ga_hbm.at[pl.ds(ep, GC)], swa).wait()
                pltpu.make_async_copy(
                    rows.at[nb], gb_hbm.at[pl.ds(ep, GC)], swb).wait()

            @pl.when(j + 1 < nch)
            def _():
                e1 = e0 + GC
                pltpu.sync_copy(dst_hbm.at[pl.ds(e1, GC)], idxd.at[nb])
                pltpu.sync_copy(src_hbm.at[pl.ds(e1, GC)], idxs.at[nb])
                pltpu.async_copy(a_hbm.at[idxd.at[nb]], rowd.at[nb], sga)
                pltpu.async_copy(b_hbm.at[idxs.at[nb]], rows.at[nb], sgb)

            pltpu.make_async_copy(a_hbm.at[idxd.at[b]], rowd.at[b], sga).wait()
            pltpu.make_async_copy(b_hbm.at[idxs.at[b]], rows.at[b], sgb).wait()
            pltpu.async_copy(rowd.at[b], ga_hbm.at[pl.ds(e0, GC)], swa)
            pltpu.async_copy(rows.at[b], gb_hbm.at[pl.ds(e0, GC)], swb)
            return carry

        lax.fori_loop(0, nch, body, 0)
        el = base + (nch - 1) * GC
        bl = (nch - 1) % 2
        pltpu.make_async_copy(rowd.at[bl], ga_hbm.at[pl.ds(el, GC)], swa).wait()
        pltpu.make_async_copy(rows.at[bl], gb_hbm.at[pl.ds(el, GC)], swb).wait()

    return k(a, b, dst, src)


def _scatter(m, dst, with_cnt):
    mesh = plsc.VectorSubcoreMesh(core_axis_name="c", subcore_axis_name="s")
    z64 = jnp.zeros((N, H), jnp.float32)
    z16 = jnp.zeros((N, 16), jnp.float32)
    ones = jnp.ones((GC, 16), jnp.float32)
    nrows = N // NS
    out_type = [jax.ShapeDtypeStruct((NC, N, H), jnp.float32)]
    if with_cnt:
        out_type.append(jax.ShapeDtypeStruct((NC, N, 16), jnp.float32))

    @functools.partial(
        pl.kernel,
        mesh=mesh,
        out_type=out_type,
        scratch_types=[
            pltpu.VMEM((2, GC), jnp.int32),
            pltpu.VMEM((2, GC, H), jnp.float32),
            pltpu.VMEM((GC, 16), jnp.float32),
            pltpu.VMEM_SHARED((N, H), jnp.float32),
            pltpu.VMEM_SHARED((N, 16), jnp.float32),
            pltpu.SemaphoreType.DMA,
            pltpu.SemaphoreType.DMA,
        ],
        compiler_params=_SC_PARAMS,
    )
    def k(m_hbm, dst_hbm, z64_hbm, z16_hbm, ones_hbm, *outs_and_scratch):
        if with_cnt:
            s_hbm, c_hbm = outs_and_scratch[:2]
            idx, rows, onev, acc, accc, si, sm = outs_and_scratch[2:]
        else:
            s_hbm = outs_and_scratch[0]
            c_hbm = None
            idx, rows, onev, acc, accc, si, sm = outs_and_scratch[1:]
        cid = lax.axis_index("c")
        sid = lax.axis_index("s")
        wid = sid * NC + cid
        base = wid * EW
        nch = EW // GC
        if with_cnt:
            pltpu.sync_copy(ones_hbm, onev)

        @pl.when(sid == 0)
        def _():
            pltpu.sync_copy(z64_hbm, acc)
            if with_cnt:
                pltpu.sync_copy(z16_hbm, accc)

        pltpu.async_copy(dst_hbm.at[pl.ds(base, GC)], idx.at[0], si)
        pltpu.async_copy(m_hbm.at[pl.ds(base, GC)], rows.at[0], sm)
        plsc.subcore_barrier()

        def body(j, carry):
            b = j % 2
            nb = 1 - b
            e0 = base + j * GC
            pltpu.make_async_copy(
                dst_hbm.at[pl.ds(e0, GC)], idx.at[b], si).wait()
            pltpu.make_async_copy(
                m_hbm.at[pl.ds(e0, GC)], rows.at[b], sm).wait()

            @pl.when(j + 1 < nch)
            def _():
                e1 = e0 + GC
                pltpu.async_copy(dst_hbm.at[pl.ds(e1, GC)], idx.at[nb], si)
                pltpu.async_copy(m_hbm.at[pl.ds(e1, GC)], rows.at[nb], sm)

            pltpu.sync_copy(rows.at[b], acc.at[idx.at[b]], add=True)
            if with_cnt:
                pltpu.sync_copy(onev, accc.at[idx.at[b]], add=True)
            return carry

        lax.fori_loop(0, nch, body, 0)
        plsc.subcore_barrier()
        r0 = sid * nrows
        pltpu.sync_copy(acc.at[pl.ds(r0, nrows)], s_hbm.at[cid, pl.ds(r0, nrows)])
        if with_cnt:
            pltpu.sync_copy(accc.at[pl.ds(r0, nrows)],
                            c_hbm.at[cid, pl.ds(r0, nrows)])

    return k(m, dst, z64, z16, ones)


def kernel(node_feature, vectors, params, edge_index):
    x0 = node_feature[0]
    src = edge_index[0, 0]
    dst = edge_index[0, 1]
    x1 = x0[:, :V]
    x1p = jnp.pad(x1, ((0, 0), (0, H - V)))
    x1h = jnp.pad(x1[:H], ((0, 0), (0, H - V)))

    x2 = x0
    c2 = None
    for li, lp in enumerate(params['gnn']):
        msg = lp['msg']
        w0 = msg['W'][0]
        w0a = w0[:D]
        w0b = jnp.zeros((D, H), jnp.float32).at[:V].set(w0[D:])
        a, b = _proj(x2, w0a, w0b)
        ga, gb = _gather(a, b, dst, src)
        m = _msgnet(ga.reshape(NE2, D2), gb.reshape(NE2, D2), msg)
        if li == 0:
            s2, c2 = _scatter(m.reshape(NE, H), dst, with_cnt=True)
        else:
            (s2,) = _scatter(m.reshape(NE, H), dst, with_cnt=False)
        x2 = _upd(x1p, x2, s2, c2, lp['upd'])

    out = _final(x2, x1h, params['pre'], params['post'])
    return out[0, :V]


# speedup vs baseline: 2.5961x; 1.1132x over previous
import functools

import jax
import jax.numpy as jnp
from jax import lax
from jax.experimental import pallas as pl
from jax.experimental.pallas import tpu as pltpu
from jax.experimental.pallas import tpu_sc as plsc

N = 10000
NE = 320000
D = 128
V = 42
H = 64
EBLK = 8000
NBLK = 2000
EPS = 1e-5


def _swish(x):
    return x * (0.5 * jnp.tanh(0.5 * x) + 0.5)


def _pad8(b):
    return jnp.zeros((8, H), jnp.float32).at[0].set(b)


def _proj_body(x_ref, wa_ref, wb_ref, a_ref, b_ref):
    x = x_ref[...]
    a_ref[...] = jnp.dot(x, wa_ref[...], preferred_element_type=jnp.float32)
    b_ref[...] = jnp.dot(x, wb_ref[...], preferred_element_type=jnp.float32)


def _proj(x2, w0a, w0b):
    g = N // NBLK
    return pl.pallas_call(
        _proj_body,
        grid=(g,),
        in_specs=[
            pl.BlockSpec((NBLK, D), lambda i: (i, 0)),
            pl.BlockSpec((D, H), lambda i: (0, 0)),
            pl.BlockSpec((D, H), lambda i: (0, 0)),
        ],
        out_specs=[
            pl.BlockSpec((NBLK, H), lambda i: (i, 0)),
            pl.BlockSpec((NBLK, H), lambda i: (i, 0)),
        ],
        out_shape=[
            jax.ShapeDtypeStruct((N, H), jnp.float32),
            jax.ShapeDtypeStruct((N, H), jnp.float32),
        ],
    )(x2, w0a, w0b)


NE2 = NE // 2
EBLK2 = EBLK
D2 = 2 * H


def _msg0_body(ga_ref, gb_ref, b0_ref, h_ref, st_ref, acc):
    i = pl.program_id(0)
    h = _swish(ga_ref[...] + gb_ref[...] + b0_ref[0:1, :])
    hb = h.astype(jnp.bfloat16)
    h_ref[...] = hb
    hf = hb.astype(jnp.float32)

    @pl.when(i == 0)
    def _():
        acc[...] = jnp.zeros_like(acc)

    acc[0:1, :] = acc[0:1, :] + jnp.sum(hf, axis=0, keepdims=True)
    acc[1:2, :] = acc[1:2, :] + jnp.sum(hf * hf, axis=0, keepdims=True)

    @pl.when(i == pl.num_programs(0) - 1)
    def _():
        st_ref[...] = acc[...]


def _msg123_body(x_in_ref, st_ref, gbe_ref, w1_ref, w2_ref, w3_ref,
                 b123_ref, psum_ref, x_ref, m_ref, acc, nrm, wsel):
    p = pl.program_id(0)
    i = pl.program_id(1)

    @pl.when(i == 0)
    def _():
        s0 = jnp.where(p == 0, st_ref[0:1, :], acc[0:1, :])
        s1 = jnp.where(p == 0, st_ref[1:2, :], acc[1:2, :])
        mu = jnp.dot(s0, psum_ref[...],
                     preferred_element_type=jnp.float32) * (1.0 / NE)
        msq = jnp.dot(s1, psum_ref[...],
                      preferred_element_type=jnp.float32) * (1.0 / NE)
        var = msq - mu * mu
        g = jnp.where(p == 0, gbe_ref[0:1, :],
                      jnp.where(p == 1, gbe_ref[2:3, :], gbe_ref[4:5, :]))
        be = jnp.where(p == 0, gbe_ref[1:2, :],
                       jnp.where(p == 1, gbe_ref[3:4, :], gbe_ref[5:6, :]))
        s = g * lax.rsqrt(var + EPS)
        nrm[0:1, :] = s
        nrm[1:2, :] = be - mu * s
        nrm[2:3, :] = jnp.where(p == 0, b123_ref[0:1, :],
                                jnp.where(p == 1, b123_ref[1:2, :],
                                          b123_ref[2:3, :]))
        wsel[...] = jnp.where(p == 0, w1_ref[...],
                              jnp.where(p == 1, w2_ref[...], w3_ref[...]))
        acc[...] = jnp.zeros_like(acc)

    xn = x_in_ref[...].astype(jnp.float32) * nrm[0:1, :] + nrm[1:2, :]
    y = _swish(
        jnp.dot(xn, wsel[...], preferred_element_type=jnp.float32)
        + nrm[2:3, :]
    )

    @pl.when(p < 2)
    def _():
        yb = y.astype(jnp.bfloat16)
        x_ref[...] = yb
        yf = yb.astype(jnp.float32)
        acc[0:1, :] = acc[0:1, :] + jnp.sum(yf, axis=0, keepdims=True)
        acc[1:2, :] = acc[1:2, :] + jnp.sum(yf * yf, axis=0, keepdims=True)

    @pl.when(p == 2)
    def _():
        m_ref[...] = y


def _dup(v):
    return jnp.concatenate([v, v])


def _pad8d(rows):
    out = jnp.zeros((8, D2), jnp.float32)
    for r, v in enumerate(rows):
        out = out.at[r].set(_dup(v))
    return out


def _bdiag(w):
    return (jnp.zeros((D2, D2), jnp.float32)
            .at[:H, :H].set(w).at[H:, H:].set(w))


def _msgnet(ga, gb, msg):
    ge = NE2 // EBLK2
    b0p = _pad8d([msg['b'][0]])
    gbe = _pad8d([msg['g'][0], msg['be'][0], msg['g'][1], msg['be'][1],
                  msg['g'][2], msg['be'][2]])
    b123 = _pad8d([msg['b'][1], msg['b'][2], msg['b'][3]])
    eye = jnp.eye(H, dtype=jnp.float32)
    psum = jnp.block([[eye, eye], [eye, eye]])
    h1, st1 = pl.pallas_call(
        _msg0_body,
        grid=(ge,),
        in_specs=[
            pl.BlockSpec((EBLK2, D2), lambda i: (i, 0)),
            pl.BlockSpec((EBLK2, D2), lambda i: (i, 0)),
            pl.BlockSpec((8, D2), lambda i: (0, 0)),
        ],
        out_specs=[
            pl.BlockSpec((EBLK2, D2), lambda i: (i, 0)),
            pl.BlockSpec((8, D2), lambda i: (0, 0)),
        ],
        out_shape=[
            jax.ShapeDtypeStruct((NE2, D2), jnp.bfloat16),
            jax.ShapeDtypeStruct((8, D2), jnp.float32),
        ],
        scratch_shapes=[pltpu.VMEM((8, D2), jnp.float32)],
    )(ga, gb, b0p)

    cst = lambda shape: pl.BlockSpec(shape, lambda p, i: tuple(0 for _ in shape))
    _, m = pl.pallas_call(
        _msg123_body,
        grid=(3, ge),
        in_specs=[
            pl.BlockSpec((EBLK2, D2), lambda p, i: (i, 0)),
            cst((8, D2)), cst((8, D2)),
            cst((D2, D2)), cst((D2, D2)), cst((D2, D2)), cst((8, D2)),
            cst((D2, D2)),
        ],
        out_specs=[
            pl.BlockSpec((EBLK2, D2), lambda p, i: (jnp.where(p < 2, i, 0), 0)),
            pl.BlockSpec((EBLK2, D2), lambda p, i: (jnp.where(p == 2, i, 0), 0)),
        ],
        out_shape=[
            jax.ShapeDtypeStruct((NE2, D2), jnp.bfloat16),
            jax.ShapeDtypeStruct((NE2, D2), jnp.float32),
        ],
        scratch_shapes=[
            pltpu.VMEM((8, D2), jnp.float32),
            pltpu.VMEM((8, D2), jnp.float32),
            pltpu.VMEM((D2, D2), jnp.float32),
        ],
        input_output_aliases={0: 0},
    )(h1, st1, gbe, _bdiag(msg['W'][1]), _bdiag(msg['W'][2]),
      _bdiag(msg['W'][3]), b123, psum)
    return m


def _upd_body(x1_ref, x2_ref, s2_ref, c2_ref,
              wa_ref, wb_ref, wc_ref, b0_ref, w1_ref, b1_ref,
              w2_ref, b2_ref, w3_ref, b3_ref, xo_ref):
    cnt = c2_ref[0, :, 0:1] + c2_ref[1, :, 0:1]
    cnt = jnp.maximum(cnt, 1.0)
    agg = (s2_ref[0] + s2_ref[1]) / cnt
    t = _swish(
        jnp.dot(x1_ref[...], wa_ref[...], preferred_element_type=jnp.float32)
        + jnp.dot(x2_ref[...], wb_ref[...], preferred_element_type=jnp.float32)
        + jnp.dot(agg, wc_ref[...], preferred_element_type=jnp.float32)
        + b0_ref[0:1, :]
    )
    t = _swish(jnp.dot(t, w1_ref[...], preferred_element_type=jnp.float32) + b1_ref[0:1, :])
    t = _swish(jnp.dot(t, w2_ref[...], preferred_element_type=jnp.float32) + b2_ref[0:1, :])
    t = _swish(jnp.dot(t, w3_ref[...], preferred_element_type=jnp.float32) + b3_ref[0:1, :])
    xo_ref[...] = x2_ref[...] + t


def _upd(x1p, x2, s2, c2, upd_p):
    wu0 = upd_p['W'][0]
    wa = jnp.zeros((H, H), jnp.float32).at[:V].set(wu0[:V])
    wb = wu0[V:V + D]
    wc = wu0[V + D:]
    g = N // NBLK
    cst = lambda shape: pl.BlockSpec(shape, lambda i: tuple(0 for _ in shape))
    return pl.pallas_call(
        _upd_body,
        grid=(g,),
        in_specs=[
            pl.BlockSpec((NBLK, H), lambda i: (i, 0)),
            pl.BlockSpec((NBLK, D), lambda i: (i, 0)),
            pl.BlockSpec((2, NBLK, H), lambda i: (0, i, 0)),
            pl.BlockSpec((2, NBLK, 16), lambda i: (0, i, 0)),
            cst((H, H)), cst((D, H)), cst((H, H)), cst((8, H)),
            cst((H, H)), cst((8, H)),
            cst((H, H)), cst((8, H)),
            cst((H, D)), cst((8, D)),
        ],
        out_specs=pl.BlockSpec((NBLK, D), lambda i: (i, 0)),
        out_shape=jax.ShapeDtypeStruct((N, D), jnp.float32),
    )(x1p, x2, s2, c2,
      wa, wb, wc, _pad8(upd_p['b'][0]),
      upd_p['W'][1], _pad8(upd_p['b'][1]),
      upd_p['W'][2], _pad8(upd_p['b'][2]),
      upd_p['W'][3],
      jnp.zeros((8, D), jnp.float32).at[0].set(upd_p['b'][3]))


def _final_body(x2_ref, x1h_ref,
                wp0_ref, bp0_ref, wp1_ref, bp1_ref, wp2_ref, bp2_ref,
                wp3_ref, bp3_ref, wq0_ref, bq0_ref, wq1_ref, bq1_ref,
                out_ref, acc):
    i = pl.program_id(0)
    t = _swish(jnp.dot(x2_ref[...], wp0_ref[...], preferred_element_type=jnp.float32) + bp0_ref[0:1, :])
    t = _swish(jnp.dot(t, wp1_ref[...], preferred_element_type=jnp.float32) + bp1_ref[0:1, :])
    t = _swish(jnp.dot(t, wp2_ref[...], preferred_element_type=jnp.float32) + bp2_ref[0:1, :])
    h = jnp.dot(t, wp3_ref[...], preferred_element_type=jnp.float32) + bp3_ref[0:1, :]

    @pl.when(i == 0)
    def _():
        acc[...] = jnp.zeros_like(acc)

    acc[0:1, :] = acc[0:1, :] + jnp.sum(h, axis=0, keepdims=True)

    @pl.when(i == pl.num_programs(0) - 1)
    def _():
        pooled = acc[...] / N
        c = _swish(jnp.dot(pooled, wq0_ref[...], preferred_element_type=jnp.float32) + bq0_ref[0:1, :])
        coeff = jnp.dot(c, wq1_ref[...], preferred_element_type=jnp.float32) + bq1_ref[0:1, :]
        out_ref[...] = jnp.dot(coeff, x1h_ref[...], preferred_element_type=jnp.float32)


def _final(x2, x1h, pre_p, post_p):
    g = N // NBLK
    cst = lambda shape: pl.BlockSpec(shape, lambda i: tuple(0 for _ in shape))
    return pl.pallas_call(
        _final_body,
        grid=(g,),
        in_specs=[
            pl.BlockSpec((NBLK, D), lambda i: (i, 0)),
            cst((H, H)),
            cst((D, H)), cst((8, H)),
            cst((H, H)), cst((8, H)),
            cst((H, H)), cst((8, H)),
            cst((H, H)), cst((8, H)),
            cst((H, H)), cst((8, H)),
            cst((H, H)), cst((8, H)),
        ],
        out_specs=pl.BlockSpec((8, H), lambda i: (0, 0)),
        out_shape=jax.ShapeDtypeStruct((8, H), jnp.float32),
        scratch_shapes=[pltpu.VMEM((8, H), jnp.float32)],
    )(x2, x1h,
      pre_p['W'][0], _pad8(pre_p['b'][0]),
      pre_p['W'][1], _pad8(pre_p['b'][1]),
      pre_p['W'][2], _pad8(pre_p['b'][2]),
      pre_p['W'][3], _pad8(pre_p['b'][3]),
      post_p['W'][0], _pad8(post_p['b'][0]),
      post_p['W'][1], _pad8(post_p['b'][1]))


NC = 2
NS = 16
NW = NC * NS
EW = NE // NW
GC = 400


_SC_PARAMS = pltpu.CompilerParams(use_tc_tiling_on_sc=False)


def _gather(a, b, dst, src):
    mesh = plsc.VectorSubcoreMesh(core_axis_name="c", subcore_axis_name="s")

    @functools.partial(
        pl.kernel,
        mesh=mesh,
        out_type=[
            jax.ShapeDtypeStruct((NE, H), jnp.float32),
            jax.ShapeDtypeStruct((NE, H), jnp.float32),
        ],
        scratch_types=[
            pltpu.VMEM((3, GC), jnp.int32),
            pltpu.VMEM((3, GC), jnp.int32),
            pltpu.VMEM((2, GC, H), jnp.float32),
            pltpu.VMEM((2, GC, H), jnp.float32),
            pltpu.SemaphoreType.DMA,
            pltpu.SemaphoreType.DMA,
            pltpu.SemaphoreType.DMA,
            pltpu.SemaphoreType.DMA,
            pltpu.SemaphoreType.DMA,
        ],
        compiler_params=_SC_PARAMS,
    )
    def k(a_hbm, b_hbm, dst_hbm, src_hbm, ga_hbm, gb_hbm, idxd, idxs,
          rowd, rows, sga, sgb, swa, swb, si):
        wid = lax.axis_index("s") * NC + lax.axis_index("c")
        base = wid * EW
        nch = EW // GC

        pltpu.async_copy(dst_hbm.at[pl.ds(base, GC)], idxd.at[0], si)
        pltpu.async_copy(src_hbm.at[pl.ds(base, GC)], idxs.at[0], si)
        pltpu.async_copy(dst_hbm.at[pl.ds(base + GC, GC)], idxd.at[1], si)
        pltpu.async_copy(src_hbm.at[pl.ds(base + GC, GC)], idxs.at[1], si)
        pltpu.make_async_copy(dst_hbm.at[pl.ds(base, GC)], idxd.at[0], si).wait()
        pltpu.make_async_copy(src_hbm.at[pl.ds(base, GC)], idxs.at[0], si).wait()
        pltpu.async_copy(a_hbm.at[idxd.at[0]], rowd.at[0], sga)
        pltpu.async_copy(b_hbm.at[idxs.at[0]], rows.at[0], sgb)

        def body(j, carry):
            b = j % 2
            nb = 1 - b
            q1 = (j + 1) % 3
            e0 = base + j * GC

            @pl.when(j >= 1)
            def _():
                ep = e0 - GC
                pltpu.make_async_copy(
                    rowd.at[nb], ga_hbm.at[pl.ds(ep, GC)], swa).wait()
                pltpu.make_async_copy(
                    rows.at[nb], gb_hbm.at[pl.ds(ep, GC)], swb).wait()

            @pl.when(j + 2 < nch)
            def _():
                e2 = e0 + 2 * GC
                q2 = (j + 2) % 3
                pltpu.async_copy(dst_hbm.at[pl.ds(e2, GC)], idxd.at[q2], si)
                pltpu.async_copy(src_hbm.at[pl.ds(e2, GC)], idxs.at[q2], si)

            @pl.when(j + 1 < nch)
            def _():
                e1 = e0 + GC
                pltpu.make_async_copy(
                    dst_hbm.at[pl.ds(e1, GC)], idxd.at[q1], si).wait()
                pltpu.make_async_copy(
                    src_hbm.at[pl.ds(e1, GC)], idxs.at[q1], si).wait()
                pltpu.async_copy(a_hbm.at[idxd.at[q1]], rowd.at[nb], sga)
                pltpu.async_copy(b_hbm.at[idxs.at[q1]], rows.at[nb], sgb)

            pltpu.make_async_copy(a_hbm.at[idxd.at[j % 3]], rowd.at[b], sga).wait()
            pltpu.make_async_copy(b_hbm.at[idxs.at[j % 3]], rows.at[b], sgb).wait()
            pltpu.async_copy(rowd.at[b], ga_hbm.at[pl.ds(e0, GC)], swa)
            pltpu.async_copy(rows.at[b], gb_hbm.at[pl.ds(e0, GC)], swb)
            return carry

        lax.fori_loop(0, nch, body, 0)
        el = base + (nch - 1) * GC
        bl = (nch - 1) % 2
        pltpu.make_async_copy(rowd.at[bl], ga_hbm.at[pl.ds(el, GC)], swa).wait()
        pltpu.make_async_copy(rows.at[bl], gb_hbm.at[pl.ds(el, GC)], swb).wait()

    return k(a, b, dst, src)


def _scatter(m, dst, with_cnt):
    mesh = plsc.VectorSubcoreMesh(core_axis_name="c", subcore_axis_name="s")
    z64 = jnp.zeros((N, H), jnp.float32)
    z16 = jnp.zeros((N, 16), jnp.float32)
    ones = jnp.ones((GC, 16), jnp.float32)
    nrows = N // NS
    out_type = [jax.ShapeDtypeStruct((NC, N, H), jnp.float32)]
    if with_cnt:
        out_type.append(jax.ShapeDtypeStruct((NC, N, 16), jnp.float32))

    @functools.partial(
        pl.kernel,
        mesh=mesh,
        out_type=out_type,
        scratch_types=[
            pltpu.VMEM((2, GC), jnp.int32),
            pltpu.VMEM((2, GC, H), jnp.float32),
            pltpu.VMEM((GC, 16), jnp.float32),
            pltpu.VMEM_SHARED((N, H), jnp.float32),
            pltpu.VMEM_SHARED((N, 16), jnp.float32),
            pltpu.SemaphoreType.DMA,
            pltpu.SemaphoreType.DMA,
        ],
        compiler_params=_SC_PARAMS,
    )
    def k(m_hbm, dst_hbm, z64_hbm, z16_hbm, ones_hbm, *outs_and_scratch):
        if with_cnt:
            s_hbm, c_hbm = outs_and_scratch[:2]
            idx, rows, onev, acc, accc, si, sm = outs_and_scratch[2:]
        else:
            s_hbm = outs_and_scratch[0]
            c_hbm = None
            idx, rows, onev, acc, accc, si, sm = outs_and_scratch[1:]
        cid = lax.axis_index("c")
        sid = lax.axis_index("s")
        wid = sid * NC + cid
        base = wid * EW
        nch = EW // GC
        if with_cnt:
            pltpu.sync_copy(ones_hbm, onev)

        @pl.when(sid == 0)
        def _():
            pltpu.sync_copy(z64_hbm, acc)
            if with_cnt:
                pltpu.sync_copy(z16_hbm, accc)

        pltpu.async_copy(dst_hbm.at[pl.ds(base, GC)], idx.at[0], si)
        pltpu.async_copy(m_hbm.at[pl.ds(base, GC)], rows.at[0], sm)
        plsc.subcore_barrier()

        def body(j, carry):
            b = j % 2
            nb = 1 - b
            e0 = base + j * GC
            pltpu.make_async_copy(
                dst_hbm.at[pl.ds(e0, GC)], idx.at[b], si).wait()
            pltpu.make_async_copy(
                m_hbm.at[pl.ds(e0, GC)], rows.at[b], sm).wait()

            @pl.when(j + 1 < nch)
            def _():
                e1 = e0 + GC
                pltpu.async_copy(dst_hbm.at[pl.ds(e1, GC)], idx.at[nb], si)
                pltpu.async_copy(m_hbm.at[pl.ds(e1, GC)], rows.at[nb], sm)

            pltpu.sync_copy(rows.at[b], acc.at[idx.at[b]], add=True)
            if with_cnt:
                pltpu.sync_copy(onev, accc.at[idx.at[b]], add=True)
            return carry

        lax.fori_loop(0, nch, body, 0)
        plsc.subcore_barrier()
        r0 = sid * nrows
        pltpu.sync_copy(acc.at[pl.ds(r0, nrows)], s_hbm.at[cid, pl.ds(r0, nrows)])
        if with_cnt:
            pltpu.sync_copy(accc.at[pl.ds(r0, nrows)],
                            c_hbm.at[cid, pl.ds(r0, nrows)])

    return k(m, dst, z64, z16, ones)


def kernel(node_feature, vectors, params, edge_index):
    x0 = node_feature[0]
    src = edge_index[0, 0]
    dst = edge_index[0, 1]
    x1 = x0[:, :V]
    x1p = jnp.pad(x1, ((0, 0), (0, H - V)))
    x1h = jnp.pad(x1[:H], ((0, 0), (0, H - V)))

    x2 = x0
    c2 = None
    for li, lp in enumerate(params['gnn']):
        msg = lp['msg']
        w0 = msg['W'][0]
        w0a = w0[:D]
        w0b = jnp.zeros((D, H), jnp.float32).at[:V].set(w0[D:])
        a, b = _proj(x2, w0a, w0b)
        ga, gb = _gather(a, b, dst, src)
        m = _msgnet(ga.reshape(NE2, D2), gb.reshape(NE2, D2), msg)
        if li == 0:
            s2, c2 = _scatter(m.reshape(NE, H), dst, with_cnt=True)
        else:
            (s2,) = _scatter(m.reshape(NE, H), dst, with_cnt=False)
        x2 = _upd(x1p, x2, s2, c2, lp['upd'])

    out = _final(x2, x1h, params['pre'], params['post'])
    return out[0, :V]
